# Initial kernel scaffold; baseline (speedup 1.0000x reference)
import jax
import jax.numpy as jnp
from jax.experimental import pallas as pl


def kernel(h, x, params, edge_index, mask_ligand):
    raise NotImplementedError("write your pallas kernel here")


# speedup vs baseline: 19.9158x; 19.9158x over previous
import functools

import jax
import jax.numpy as jnp
import numpy as np
from jax import lax
from jax.experimental import pallas as pl
from jax.experimental.pallas import tpu as pltpu
from jax.experimental.pallas import tpu_sc as plsc

NH = 16
HD = 8
HID = 128
NG = 20
RF = NG * 4
R_MAX = 10.0
_GSTEP = R_MAX / (NG - 1)
_GCOEFF = -0.5 / _GSTEP ** 2
_ISQ = 1.0 / float(np.sqrt(HD))

_TN = 1000
_TE = 512
_SCW = 128


def _ln_relu(y, g, b):
    mu = jnp.mean(y, -1, keepdims=True)
    var = jnp.mean((y - mu) ** 2, -1, keepdims=True)
    return jnp.maximum((y - mu) * lax.rsqrt(var + 1e-5) * g + b, 0.0)


def _mlp(xv, W1, b1, g, bln, W2, b2):
    y = jnp.dot(xv, W1, preferred_element_type=jnp.float32) + b1
    y = _ln_relu(y, g, bln)
    return jnp.dot(y, W2, preferred_element_type=jnp.float32) + b2


def _node1_body(h_ref, xm_ref, Wq1, bq1, gq, blq, Wq2, bq2,
                WdP, bPd, WsP, tda_ref, tdb_ref, tsrc_ref):
    h = h_ref[...]
    q1 = _mlp(h, Wq1[...], bq1[...], gq[...], blq[...], Wq2[...], bq2[...])
    Pd = jnp.dot(h, WdP[...], preferred_element_type=jnp.float32) + bPd[...]
    Ps = jnp.dot(h, WsP[...], preferred_element_type=jnp.float32)
    xm = xm_ref[...]
    z = jnp.zeros((h.shape[0], 112), jnp.float32)
    tda_ref[...] = Pd
    tdb_ref[...] = jnp.concatenate([q1, xm, z], axis=-1)
    tsrc_ref[...] = jnp.concatenate([Ps, xm, z], axis=-1)


def _edge_common(gdb, gs):
    relx = gdb[:, 128:129] - gs[:, 256:257]
    rely = gdb[:, 129:130] - gs[:, 257:258]
    relz = gdb[:, 130:131] - gs[:, 258:259]
    mi = gdb[:, 131:132]
    mj = gs[:, 259:260]
    d = jnp.sqrt(relx * relx + rely * rely + relz * relz + 1e-12)
    offs = lax.broadcasted_iota(jnp.int32, (d.shape[0], NG), 1).astype(jnp.float32) * _GSTEP
    gauss = jnp.exp(_GCOEFF * (d - offs) ** 2)
    oh = (
        (1.0 - mj) * (1.0 - mi),
        (1.0 - mj) * mi,
        mj * (1.0 - mi),
        mj * mi,
    )
    return (relx, rely, relz), gauss, oh


def _edge_attn(gda, gdb, gs, Wcat, gk, bk, W2k, b2k, gv, bv, W2v, b2v, ewb, HS, EX):
    rel, gauss, oh = _edge_common(gdb, gs)
    Gr = jnp.dot(gauss, Wcat, preferred_element_type=jnp.float32)
    racc_k = sum(oh[t] * Gr[:, t * 128:(t + 1) * 128] for t in range(4))
    racc_v = sum(oh[t] * Gr[:, 512 + t * 128:512 + (t + 1) * 128] for t in range(4))
    racc_e = sum(oh[t] * Gr[:, 1024 + t:1025 + t] for t in range(4))
    y_k = racc_k + gda[:, 0:128] + gs[:, 0:128]
    y_v = racc_v + gda[:, 128:256] + gs[:, 128:256]
    k_ = jnp.dot(_ln_relu(y_k, gk, bk), W2k, preferred_element_type=jnp.float32) + b2k
    v_ = jnp.dot(_ln_relu(y_v, gv, bv), W2v, preferred_element_type=jnp.float32) + b2v
    e_w = 1.0 / (1.0 + jnp.exp(-(racc_e + ewb)))
    q = gdb[:, 0:128]
    logits = jnp.dot(q * k_, HS, preferred_element_type=jnp.float32) * _ISQ
    expl = jnp.exp(logits)
    return rel, k_, v_, e_w, expl


def _edge1_body(gda_ref, gdb_ref, gs_ref, Wcat, gk, bk, W2k, b2k, gv, bv, W2v, b2v,
                ewb, HS, EX, wv_ref, ex_ref):
    _, _, v_, e_w, expl = _edge_attn(
        gda_ref[...], gdb_ref[...], gs_ref[...], Wcat[...], gk[...], bk[...], W2k[...],
        b2k[...], gv[...], bv[...], W2v[...], b2v[...], ewb[...], HS[...], EX[...])
    expl_x = jnp.dot(expl, EX[...], preferred_element_type=jnp.float32)
    wv_ref[...] = (v_ * e_w) * expl_x
    ex_ref[...] = expl_x


def _edge2_body(gda_ref, gdb_ref, gs_ref, Wcat, gk, bk, W2k, b2k, gv, bv, W2v, b2v,
                ewb, HS, EX, rows_ref):
    (relx, rely, relz), _, v2, ew2, expl2 = _edge_attn(
        gda_ref[...], gdb_ref[...], gs_ref[...], Wcat[...], gk[...], bk[...], W2k[...],
        b2k[...], gv[...], bv[...], W2v[...], b2v[...], ewb[...], HS[...], EX[...])
    w2 = expl2 * (v2 * ew2)
    z = jnp.zeros((w2.shape[0], 64), jnp.float32)
    rows_ref[...] = jnp.concatenate(
        [w2 * relx, w2 * rely, w2 * relz, expl2, z], axis=-1)


def _node2_body(Sn_ref, Sd_ref, h_ref, xm_ref,
                W1noA, W1noB, b1no, gno, blno, W2no, b2no,
                Wq1, bq1, gq, blq, Wq2, bq2, WdP, bPd, WsP,
                hnew_ref, tda_ref, tdb_ref, tsrc_ref):
    h = h_ref[...]
    num = Sn_ref[0] + Sn_ref[1]
    den = Sd_ref[0] + Sd_ref[1]
    out_attn = num / (den + 1e-16)
    y = (jnp.dot(out_attn, W1noA[...], preferred_element_type=jnp.float32)
         + jnp.dot(h, W1noB[...], preferred_element_type=jnp.float32) + b1no[...])
    out = jnp.dot(_ln_relu(y, gno[...], blno[...]), W2no[...],
                  preferred_element_type=jnp.float32) + b2no[...]
    h_new = out + h
    hnew_ref[...] = h_new
    q2 = _mlp(h_new, Wq1[...], bq1[...], gq[...], blq[...], Wq2[...], bq2[...])
    Pd = jnp.dot(h_new, WdP[...], preferred_element_type=jnp.float32) + bPd[...]
    Ps = jnp.dot(h_new, WsP[...], preferred_element_type=jnp.float32)
    xm = xm_ref[...]
    z = jnp.zeros((h.shape[0], 112), jnp.float32)
    tda_ref[...] = Pd
    tdb_ref[...] = jnp.concatenate([q2, xm, z], axis=-1)
    tsrc_ref[...] = jnp.concatenate([Ps, xm, z], axis=-1)


def _node3_body(U_ref, xm_ref, xout_ref):
    U = U_ref[0] + U_ref[1]
    inv = 1.0 / (U[:, 48:64] + 1e-16)
    c = [jnp.mean(U[:, 16 * t:16 * t + 16] * inv, -1, keepdims=True)
         for t in range(3)]
    xm = xm_ref[...]
    mf = xm[:, 3:4]
    xnew = xm[:, 0:3] + jnp.concatenate(c, axis=-1) * mf
    pad = jnp.zeros((xnew.shape[0], 13), jnp.float32)
    xout_ref[...] = jnp.concatenate([xnew, pad], axis=-1)


def _full(a):
    return pl.BlockSpec(a.shape, lambda i: tuple(0 for _ in a.shape))


def _call_node1(h, xm, ws):
    n = h.shape[0]
    grid = (n // _TN,)
    ins = [pl.BlockSpec((_TN, 128), lambda i: (i, 0)),
           pl.BlockSpec((_TN, 16), lambda i: (i, 0))] + [_full(w) for w in ws]
    outs = [pl.BlockSpec((_TN, 256), lambda i: (i, 0)),
            pl.BlockSpec((_TN, 256), lambda i: (i, 0)),
            pl.BlockSpec((_TN, 384), lambda i: (i, 0))]
    return pl.pallas_call(
        _node1_body, grid=grid, in_specs=ins, out_specs=outs,
        out_shape=[jax.ShapeDtypeStruct((n, 256), jnp.float32),
                   jax.ShapeDtypeStruct((n, 256), jnp.float32),
                   jax.ShapeDtypeStruct((n, 384), jnp.float32)],
    )(h, xm, *ws)


def _call_edge(body, GdA, GdB, Gs, ws, out_widths):
    e = GdA.shape[0]
    grid = (e // _TE,)
    ins = [pl.BlockSpec((_TE, 256), lambda i: (i, 0)),
           pl.BlockSpec((_TE, 256), lambda i: (i, 0)),
           pl.BlockSpec((_TE, 384), lambda i: (i, 0))] + [_full(w) for w in ws]
    outs = [pl.BlockSpec((_TE, w), lambda i: (i, 0)) for w in out_widths]
    shapes = [jax.ShapeDtypeStruct((e, w), jnp.float32) for w in out_widths]
    if len(out_widths) == 1:
        outs, shapes = outs[0], shapes[0]
    return pl.pallas_call(
        body, grid=grid, in_specs=ins, out_specs=outs, out_shape=shapes,
    )(GdA, GdB, Gs, *ws)


def _call_node2(Sn, Sd, h, xm, ws):
    n = h.shape[0]
    grid = (n // _TN,)
    ins = [pl.BlockSpec((2, _TN, 128), lambda i: (0, i, 0)),
           pl.BlockSpec((2, _TN, 128), lambda i: (0, i, 0)),
           pl.BlockSpec((_TN, 128), lambda i: (i, 0)),
           pl.BlockSpec((_TN, 16), lambda i: (i, 0))] + [_full(w) for w in ws]
    outs = [pl.BlockSpec((_TN, 128), lambda i: (i, 0)),
            pl.BlockSpec((_TN, 256), lambda i: (i, 0)),
            pl.BlockSpec((_TN, 256), lambda i: (i, 0)),
            pl.BlockSpec((_TN, 384), lambda i: (i, 0))]
    return pl.pallas_call(
        _node2_body, grid=grid, in_specs=ins, out_specs=outs,
        out_shape=[jax.ShapeDtypeStruct((n, 128), jnp.float32),
                   jax.ShapeDtypeStruct((n, 256), jnp.float32),
                   jax.ShapeDtypeStruct((n, 256), jnp.float32),
                   jax.ShapeDtypeStruct((n, 384), jnp.float32)],
    )(Sn, Sd, h, xm, *ws)


def _call_node3(U, xm):
    n = xm.shape[0]
    grid = (n // _TN,)
    ins = [pl.BlockSpec((2, _TN, 128), lambda i: (0, i, 0)),
           pl.BlockSpec((_TN, 16), lambda i: (i, 0))]
    outs = pl.BlockSpec((_TN, 16), lambda i: (i, 0))
    return pl.pallas_call(
        _node3_body, grid=grid, in_specs=ins, out_specs=outs,
        out_shape=jax.ShapeDtypeStruct((n, 16), jnp.float32),
    )(U, xm)


def _sc_gather(table, idx):
    eg = idx.shape[0]
    d = table.shape[1]
    w = 64 if d > 384 else _SCW
    mesh = plsc.VectorSubcoreMesh(core_axis_name="core", subcore_axis_name="subcore")

    @functools.partial(
        pl.kernel,
        out_type=jax.ShapeDtypeStruct((eg, d), table.dtype),
        mesh=mesh)
    def k(t_hbm, i_hbm, o_hbm):
        def body(i_vmem, o_vmem):
            pltpu.sync_copy(t_hbm.at[i_vmem.at[0]], o_vmem)

        pltpu.emit_pipeline(
            body,
            grid=(eg // w,),
            in_specs=[pl.BlockSpec((1, w), lambda i: (0, i))],
            out_specs=[pl.BlockSpec((w, d), lambda i: (i, 0))],
            core_axis_name=("core", "subcore"),
            dimension_semantics=(pltpu.PARALLEL,),
        )(i_hbm, o_hbm)

    return k(table, idx.reshape(1, eg))


def _sc_scatter_add(rows_list, idx, num_nodes):
    nrow = idx.shape[0] // _SCW
    dims = [r.shape[1] for r in rows_list]
    base, rem = nrow // 32, nrow % 32
    rpt = (num_nodes // 16) & ~7
    tail = num_nodes - 16 * rpt
    mesh = plsc.VectorSubcoreMesh(core_axis_name="core", subcore_axis_name="subcore")
    zeros = [jnp.zeros((num_nodes, d), jnp.float32) for d in dims]

    scratch = [pltpu.VMEM((_SCW,), jnp.int32)]
    for d in dims:
        scratch.append(pltpu.VMEM((_SCW, d), jnp.float32))
        scratch.append(pltpu.VMEM_SHARED((num_nodes, d), jnp.float32))

    @functools.partial(
        pl.kernel,
        out_type=[jax.ShapeDtypeStruct((2, num_nodes, d), jnp.float32)
                  for d in dims],
        mesh=mesh,
        scratch_types=scratch)
    def k(*refs):
        nz = len(dims)
        z_hbm = refs[0:nz]
        r_hbm = refs[nz:2 * nz]
        i_hbm = refs[2 * nz]
        o_hbm = refs[2 * nz + 1:3 * nz + 1]
        idxb = refs[3 * nz + 1]
        bufs = refs[3 * nz + 2::2]
        shareds = refs[3 * nz + 3::2]

        cid = lax.axis_index("core")
        sid = lax.axis_index("subcore")
        wid = sid * 2 + cid

        @pl.when(sid == 0)
        def _():
            for z, sh in zip(z_hbm, shareds):
                pltpu.sync_copy(z, sh)

        plsc.subcore_barrier()

        start = wid * base + jnp.minimum(wid, rem)
        cnt = base + jnp.where(wid < rem, 1, 0)

        @pl.loop(0, cnt)
        def _(t):
            off = pl.multiple_of((start + t) * _SCW, _SCW)
            pltpu.sync_copy(i_hbm.at[pl.ds(off, _SCW)], idxb)
            for r, buf, sh in zip(r_hbm, bufs, shareds):
                pltpu.sync_copy(r.at[pl.ds(off, _SCW)], buf)
                pltpu.sync_copy(buf, sh.at[idxb], add=True)

        plsc.subcore_barrier()
        ns = pl.multiple_of(sid * rpt, 8)
        for sh, o in zip(shareds, o_hbm):
            pltpu.sync_copy(sh.at[pl.ds(ns, rpt)], o.at[cid, pl.ds(ns, rpt)])
        if tail:
            @pl.when(sid == 0)
            def _():
                for sh, o in zip(shareds, o_hbm):
                    pltpu.sync_copy(sh.at[pl.ds(16 * rpt, tail)],
                                    o.at[cid, pl.ds(16 * rpt, tail)])

    return k(*zeros, *rows_list, idx)


def _r_weight(W1):
    return jnp.concatenate([W1[t:RF:4, :] for t in range(4)], axis=1)


def _row(v):
    return v.reshape(1, -1)


def _prep_attn_weights(pk, pv, pew):
    Wcat = jnp.concatenate(
        [_r_weight(pk['W1']), _r_weight(pv['W1']),
         jnp.concatenate([pew['W'][t:RF:4, :] for t in range(4)], axis=1)],
        axis=1)
    HS = (jnp.arange(128)[:, None] // HD
          == jnp.arange(NH)[None, :]).astype(jnp.float32)
    EX = HS.T
    return [Wcat, _row(pk['g']), _row(pk['bln']), pk['W2'], _row(pk['b2']),
            _row(pv['g']), _row(pv['bln']), pv['W2'], _row(pv['b2']),
            pew['b'].reshape(1, 1), HS, EX]


def _proj_weights(pk, pv):
    WdP = jnp.concatenate([pk['W1'][RF:RF + HID], pv['W1'][RF:RF + HID]], axis=1)
    bPd = _row(jnp.concatenate([pk['b1'], pv['b1']]))
    WsP = jnp.concatenate([pk['W1'][RF + HID:], pv['W1'][RF + HID:]], axis=1)
    return WdP, bPd, WsP


def _mlp_weights(p):
    return [p['W1'], _row(p['b1']), _row(p['g']), _row(p['bln']),
            p['W2'], _row(p['b2'])]


def kernel(h, x, params, edge_index, mask_ligand):
    n = h.shape[0]
    e = edge_index.shape[1]
    src = edge_index[0].astype(jnp.int32)
    dst = edge_index[1].astype(jnp.int32)
    mf = mask_ligand.astype(jnp.float32)
    xm = jnp.concatenate(
        [x, mf[:, None], jnp.zeros((n, 12), jnp.float32)], axis=-1)

    WdP1, bPd1, WsP1 = _proj_weights(params['hk'], params['hv'])
    tda1, tdb1, tsrc1 = _call_node1(
        h, xm, _mlp_weights(params['hq']) + [WdP1, bPd1, WsP1])
    GdA1 = _sc_gather(tda1, dst)
    GdB1 = _sc_gather(tdb1, dst)
    Gs1 = _sc_gather(tsrc1, src)
    ew1 = _prep_attn_weights(params['hk'], params['hv'], params['ew_h'])
    wv, ex = _call_edge(_edge1_body, GdA1, GdB1, Gs1, ew1, [128, 128])
    (Sn,) = _sc_scatter_add([wv], dst, n)
    (Sd,) = _sc_scatter_add([ex], dst, n)

    pno = params['node_out']
    WdP2, bPd2, WsP2 = _proj_weights(params['xk'], params['xv'])
    ws2 = [pno['W1'][:HID], pno['W1'][HID:], _row(pno['b1']), _row(pno['g']),
           _row(pno['bln']), pno['W2'], _row(pno['b2'])] \
        + _mlp_weights(params['xq']) + [WdP2, bPd2, WsP2]
    h_new, tda2, tdb2, tsrc2 = _call_node2(Sn, Sd, h, xm, ws2)

    GdA2 = _sc_gather(tda2, dst)
    GdB2 = _sc_gather(tdb2, dst)
    Gs2 = _sc_gather(tsrc2, src)
    ew2 = _prep_attn_weights(params['xk'], params['xv'], params['ew_x'])
    rows2 = _call_edge(_edge2_body, GdA2, GdB2, Gs2, ew2, [128])
    (U,) = _sc_scatter_add([rows2], dst, n)
    xout = _call_node3(U, xm)
    return h_new, xout[:, :3]


# speedup vs baseline: 24.2356x; 1.2169x over previous
import functools

import jax
import jax.numpy as jnp
import numpy as np
from jax import lax
from jax.experimental import pallas as pl
from jax.experimental.pallas import tpu as pltpu
from jax.experimental.pallas import tpu_sc as plsc

NH = 16
HD = 8
HID = 128
NG = 20
RF = NG * 4
R_MAX = 10.0
_GSTEP = R_MAX / (NG - 1)
_GCOEFF = -0.5 / _GSTEP ** 2
_ISQ = 1.0 / float(np.sqrt(HD))

_TN = 1000
_TE = 640
_SCW = 128


def _ln_relu(y, g, b):
    mu = jnp.mean(y, -1, keepdims=True)
    var = jnp.mean((y - mu) ** 2, -1, keepdims=True)
    return jnp.maximum((y - mu) * lax.rsqrt(var + 1e-5) * g + b, 0.0)


def _mlp(xv, W1, b1, g, bln, W2, b2):
    y = jnp.dot(xv, W1, preferred_element_type=jnp.float32) + b1
    y = _ln_relu(y, g, bln)
    return jnp.dot(y, W2, preferred_element_type=jnp.float32) + b2


def _node1_body(h_ref, xm_ref, Wq1, bq1, gq, blq, Wq2, bq2,
                WdP, bPd, WsP, tda_ref, tdb_ref, tsrc_ref):
    h = h_ref[...]
    q1 = _mlp(h, Wq1[...], bq1[...], gq[...], blq[...], Wq2[...], bq2[...])
    Pd = jnp.dot(h, WdP[...], preferred_element_type=jnp.float32) + bPd[...]
    Ps = jnp.dot(h, WsP[...], preferred_element_type=jnp.float32)
    xm = xm_ref[...]
    z = jnp.zeros((h.shape[0], 112), jnp.float32)
    tda_ref[...] = Pd
    tdb_ref[...] = jnp.concatenate([q1, xm, z], axis=-1)
    tsrc_ref[...] = jnp.concatenate([Ps, xm, z], axis=-1)


def _edge_common(gdb, gs):
    relx = gdb[:, 128:129] - gs[:, 256:257]
    rely = gdb[:, 129:130] - gs[:, 257:258]
    relz = gdb[:, 130:131] - gs[:, 258:259]
    mi = gdb[:, 131:132]
    mj = gs[:, 259:260]
    d = jnp.sqrt(relx * relx + rely * rely + relz * relz + 1e-12)
    offs = lax.broadcasted_iota(jnp.int32, (d.shape[0], NG), 1).astype(jnp.float32) * _GSTEP
    gauss = jnp.exp(_GCOEFF * (d - offs) ** 2)
    oh = (
        (1.0 - mj) * (1.0 - mi),
        (1.0 - mj) * mi,
        mj * (1.0 - mi),
        mj * mi,
    )
    return (relx, rely, relz), gauss, oh


def _edge_attn(gda, gdb, gs, Wcat, gk, bk, W2k, b2k, gv, bv, W2v, b2v, ewb, HS, EX):
    rel, gauss, oh = _edge_common(gdb, gs)
    Gr = jnp.dot(gauss, Wcat, preferred_element_type=jnp.float32)
    racc_k = sum(oh[t] * Gr[:, t * 128:(t + 1) * 128] for t in range(4))
    racc_v = sum(oh[t] * Gr[:, 512 + t * 128:512 + (t + 1) * 128] for t in range(4))
    racc_e = sum(oh[t] * Gr[:, 1024 + t:1025 + t] for t in range(4))
    y_k = racc_k + gda[:, 0:128] + gs[:, 0:128]
    y_v = racc_v + gda[:, 128:256] + gs[:, 128:256]
    k_ = jnp.dot(_ln_relu(y_k, gk, bk), W2k, preferred_element_type=jnp.float32) + b2k
    v_ = jnp.dot(_ln_relu(y_v, gv, bv), W2v, preferred_element_type=jnp.float32) + b2v
    e_w = 1.0 / (1.0 + jnp.exp(-(racc_e + ewb)))
    q = gdb[:, 0:128]
    logits = jnp.dot(q * k_, HS, preferred_element_type=jnp.float32) * _ISQ
    expl = jnp.exp(logits)
    return rel, k_, v_, e_w, expl


def _edge1_body(gda_ref, gdb_ref, gs_ref, Wcat, gk, bk, W2k, b2k, gv, bv, W2v, b2v,
                ewb, HS, EX, wv_ref, ex_ref):
    _, _, v_, e_w, expl = _edge_attn(
        gda_ref[...], gdb_ref[...], gs_ref[...], Wcat[...], gk[...], bk[...], W2k[...],
        b2k[...], gv[...], bv[...], W2v[...], b2v[...], ewb[...], HS[...], EX[...])
    expl_x = jnp.dot(expl, EX[...], preferred_element_type=jnp.float32)
    wv_ref[...] = (v_ * e_w) * expl_x
    ex_ref[...] = expl_x


def _edge2_body(gda_ref, gdb_ref, gs_ref, Wcat, gk, bk, W2k, b2k, gv, bv, W2v, b2v,
                ewb, HS, EX, rows_ref):
    (relx, rely, relz), _, v2, ew2, expl2 = _edge_attn(
        gda_ref[...], gdb_ref[...], gs_ref[...], Wcat[...], gk[...], bk[...], W2k[...],
        b2k[...], gv[...], bv[...], W2v[...], b2v[...], ewb[...], HS[...], EX[...])
    w2 = expl2 * (v2 * ew2)
    z = jnp.zeros((w2.shape[0], 64), jnp.float32)
    rows_ref[...] = jnp.concatenate(
        [w2 * relx, w2 * rely, w2 * relz, expl2, z], axis=-1)


def _node2_body(Sn1_ref, Sn2_ref, Sd1_ref, Sd2_ref, h_ref, xm_ref,
                W1noA, W1noB, b1no, gno, blno, W2no, b2no,
                Wq1, bq1, gq, blq, Wq2, bq2, WdP, bPd, WsP,
                hnew_ref, tda_ref, tdb_ref, tsrc_ref):
    h = h_ref[...]
    num = Sn1_ref[0] + Sn1_ref[1] + Sn2_ref[0] + Sn2_ref[1]
    den = Sd1_ref[0] + Sd1_ref[1] + Sd2_ref[0] + Sd2_ref[1]
    out_attn = num / (den + 1e-16)
    y = (jnp.dot(out_attn, W1noA[...], preferred_element_type=jnp.float32)
         + jnp.dot(h, W1noB[...], preferred_element_type=jnp.float32) + b1no[...])
    out = jnp.dot(_ln_relu(y, gno[...], blno[...]), W2no[...],
                  preferred_element_type=jnp.float32) + b2no[...]
    h_new = out + h
    hnew_ref[...] = h_new
    q2 = _mlp(h_new, Wq1[...], bq1[...], gq[...], blq[...], Wq2[...], bq2[...])
    Pd = jnp.dot(h_new, WdP[...], preferred_element_type=jnp.float32) + bPd[...]
    Ps = jnp.dot(h_new, WsP[...], preferred_element_type=jnp.float32)
    xm = xm_ref[...]
    z = jnp.zeros((h.shape[0], 112), jnp.float32)
    tda_ref[...] = Pd
    tdb_ref[...] = jnp.concatenate([q2, xm, z], axis=-1)
    tsrc_ref[...] = jnp.concatenate([Ps, xm, z], axis=-1)


def _node3_body(U1_ref, U2_ref, xm_ref, xout_ref):
    U = U1_ref[0] + U1_ref[1] + U2_ref[0] + U2_ref[1]
    inv = 1.0 / (U[:, 48:64] + 1e-16)
    c = [jnp.mean(U[:, 16 * t:16 * t + 16] * inv, -1, keepdims=True)
         for t in range(3)]
    xm = xm_ref[...]
    mf = xm[:, 3:4]
    xnew = xm[:, 0:3] + jnp.concatenate(c, axis=-1) * mf
    pad = jnp.zeros((xnew.shape[0], 13), jnp.float32)
    xout_ref[...] = jnp.concatenate([xnew, pad], axis=-1)


def _full(a):
    return pl.BlockSpec(a.shape, lambda i: tuple(0 for _ in a.shape))


def _call_node1(h, xm, ws):
    n = h.shape[0]
    grid = (n // _TN,)
    ins = [pl.BlockSpec((_TN, 128), lambda i: (i, 0)),
           pl.BlockSpec((_TN, 16), lambda i: (i, 0))] + [_full(w) for w in ws]
    outs = [pl.BlockSpec((_TN, 256), lambda i: (i, 0)),
            pl.BlockSpec((_TN, 256), lambda i: (i, 0)),
            pl.BlockSpec((_TN, 384), lambda i: (i, 0))]
    return pl.pallas_call(
        _node1_body, grid=grid, in_specs=ins, out_specs=outs,
        out_shape=[jax.ShapeDtypeStruct((n, 256), jnp.float32),
                   jax.ShapeDtypeStruct((n, 256), jnp.float32),
                   jax.ShapeDtypeStruct((n, 384), jnp.float32)],
    )(h, xm, *ws)


def _call_edge(body, GdA, GdB, Gs, ws, out_widths):
    e = GdA.shape[0]
    grid = (e // _TE,)
    ins = [pl.BlockSpec((_TE, 256), lambda i: (i, 0)),
           pl.BlockSpec((_TE, 256), lambda i: (i, 0)),
           pl.BlockSpec((_TE, 384), lambda i: (i, 0))] + [_full(w) for w in ws]
    outs = [pl.BlockSpec((_TE, w), lambda i: (i, 0)) for w in out_widths]
    shapes = [jax.ShapeDtypeStruct((e, w), jnp.float32) for w in out_widths]
    if len(out_widths) == 1:
        outs, shapes = outs[0], shapes[0]
    return pl.pallas_call(
        body, grid=grid, in_specs=ins, out_specs=outs, out_shape=shapes,
    )(GdA, GdB, Gs, *ws)


def _call_node2(Sn1, Sn2, Sd1, Sd2, h, xm, ws):
    n = h.shape[0]
    grid = (n // _TN,)
    ins = [pl.BlockSpec((2, _TN, 128), lambda i: (0, i, 0)),
           pl.BlockSpec((2, _TN, 128), lambda i: (0, i, 0)),
           pl.BlockSpec((2, _TN, 128), lambda i: (0, i, 0)),
           pl.BlockSpec((2, _TN, 128), lambda i: (0, i, 0)),
           pl.BlockSpec((_TN, 128), lambda i: (i, 0)),
           pl.BlockSpec((_TN, 16), lambda i: (i, 0))] + [_full(w) for w in ws]
    outs = [pl.BlockSpec((_TN, 128), lambda i: (i, 0)),
            pl.BlockSpec((_TN, 256), lambda i: (i, 0)),
            pl.BlockSpec((_TN, 256), lambda i: (i, 0)),
            pl.BlockSpec((_TN, 384), lambda i: (i, 0))]
    return pl.pallas_call(
        _node2_body, grid=grid, in_specs=ins, out_specs=outs,
        out_shape=[jax.ShapeDtypeStruct((n, 128), jnp.float32),
                   jax.ShapeDtypeStruct((n, 256), jnp.float32),
                   jax.ShapeDtypeStruct((n, 256), jnp.float32),
                   jax.ShapeDtypeStruct((n, 384), jnp.float32)],
    )(Sn1, Sn2, Sd1, Sd2, h, xm, *ws)


def _call_node3(U1, U2, xm):
    n = xm.shape[0]
    grid = (n // _TN,)
    ins = [pl.BlockSpec((2, _TN, 128), lambda i: (0, i, 0)),
           pl.BlockSpec((2, _TN, 128), lambda i: (0, i, 0)),
           pl.BlockSpec((_TN, 16), lambda i: (i, 0))]
    outs = pl.BlockSpec((_TN, 16), lambda i: (i, 0))
    return pl.pallas_call(
        _node3_body, grid=grid, in_specs=ins, out_specs=outs,
        out_shape=jax.ShapeDtypeStruct((n, 16), jnp.float32),
    )(U1, U2, xm)


def _sc_gather(table, idx):
    eg = idx.shape[0]
    d = table.shape[1]
    w = 64 if d > 384 else _SCW
    mesh = plsc.VectorSubcoreMesh(core_axis_name="core", subcore_axis_name="subcore")

    @functools.partial(
        pl.kernel,
        out_type=jax.ShapeDtypeStruct((eg, d), table.dtype),
        mesh=mesh)
    def k(t_hbm, i_hbm, o_hbm):
        def body(i_vmem, o_vmem):
            pltpu.sync_copy(t_hbm.at[i_vmem.at[0]], o_vmem)

        pltpu.emit_pipeline(
            body,
            grid=(eg // w,),
            in_specs=[pl.BlockSpec((1, w), lambda i: (0, i))],
            out_specs=[pl.BlockSpec((w, d), lambda i: (i, 0))],
            core_axis_name=("core", "subcore"),
            dimension_semantics=(pltpu.PARALLEL,),
        )(i_hbm, o_hbm)

    return k(table, idx.reshape(1, eg))


def _sc_scatter_add(rows_list, idx, num_nodes):
    nrow = idx.shape[0] // _SCW
    dims = [r.shape[1] for r in rows_list]
    base, rem = nrow // 32, nrow % 32
    rpt = (num_nodes // 16) & ~7
    tail = num_nodes - 16 * rpt
    mesh = plsc.VectorSubcoreMesh(core_axis_name="core", subcore_axis_name="subcore")
    zeros = [jnp.zeros((num_nodes, d), jnp.float32) for d in dims]

    scratch = [pltpu.VMEM((_SCW,), jnp.int32)]
    for d in dims:
        scratch.append(pltpu.VMEM((_SCW, d), jnp.float32))
        scratch.append(pltpu.VMEM_SHARED((num_nodes, d), jnp.float32))

    @functools.partial(
        pl.kernel,
        out_type=[jax.ShapeDtypeStruct((2, num_nodes, d), jnp.float32)
                  for d in dims],
        mesh=mesh,
        scratch_types=scratch)
    def k(*refs):
        nz = len(dims)
        z_hbm = refs[0:nz]
        r_hbm = refs[nz:2 * nz]
        i_hbm = refs[2 * nz]
        o_hbm = refs[2 * nz + 1:3 * nz + 1]
        idxb = refs[3 * nz + 1]
        bufs = refs[3 * nz + 2::2]
        shareds = refs[3 * nz + 3::2]

        cid = lax.axis_index("core")
        sid = lax.axis_index("subcore")
        wid = sid * 2 + cid

        @pl.when(sid == 0)
        def _():
            for z, sh in zip(z_hbm, shareds):
                pltpu.sync_copy(z, sh)

        plsc.subcore_barrier()

        start = wid * base + jnp.minimum(wid, rem)
        cnt = base + jnp.where(wid < rem, 1, 0)

        @pl.loop(0, cnt)
        def _(t):
            off = pl.multiple_of((start + t) * _SCW, _SCW)
            pltpu.sync_copy(i_hbm.at[pl.ds(off, _SCW)], idxb)
            for r, buf, sh in zip(r_hbm, bufs, shareds):
                pltpu.sync_copy(r.at[pl.ds(off, _SCW)], buf)
                pltpu.sync_copy(buf, sh.at[idxb], add=True)

        plsc.subcore_barrier()
        ns = pl.multiple_of(sid * rpt, 8)
        for sh, o in zip(shareds, o_hbm):
            pltpu.sync_copy(sh.at[pl.ds(ns, rpt)], o.at[cid, pl.ds(ns, rpt)])
        if tail:
            @pl.when(sid == 0)
            def _():
                for sh, o in zip(shareds, o_hbm):
                    pltpu.sync_copy(sh.at[pl.ds(16 * rpt, tail)],
                                    o.at[cid, pl.ds(16 * rpt, tail)])

    return k(*zeros, *rows_list, idx)


def _r_weight(W1):
    return jnp.concatenate([W1[t:RF:4, :] for t in range(4)], axis=1)


def _row(v):
    return v.reshape(1, -1)


def _prep_attn_weights(pk, pv, pew):
    Wcat = jnp.concatenate(
        [_r_weight(pk['W1']), _r_weight(pv['W1']),
         jnp.concatenate([pew['W'][t:RF:4, :] for t in range(4)], axis=1)],
        axis=1)
    HS = (jnp.arange(128)[:, None] // HD
          == jnp.arange(NH)[None, :]).astype(jnp.float32)
    EX = HS.T
    return [Wcat, _row(pk['g']), _row(pk['bln']), pk['W2'], _row(pk['b2']),
            _row(pv['g']), _row(pv['bln']), pv['W2'], _row(pv['b2']),
            pew['b'].reshape(1, 1), HS, EX]


def _proj_weights(pk, pv):
    WdP = jnp.concatenate([pk['W1'][RF:RF + HID], pv['W1'][RF:RF + HID]], axis=1)
    bPd = _row(jnp.concatenate([pk['b1'], pv['b1']]))
    WsP = jnp.concatenate([pk['W1'][RF + HID:], pv['W1'][RF + HID:]], axis=1)
    return WdP, bPd, WsP


def _mlp_weights(p):
    return [p['W1'], _row(p['b1']), _row(p['g']), _row(p['bln']),
            p['W2'], _row(p['b2'])]


def kernel(h, x, params, edge_index, mask_ligand):
    n = h.shape[0]
    e = edge_index.shape[1]
    src = edge_index[0].astype(jnp.int32)
    dst = edge_index[1].astype(jnp.int32)
    mf = mask_ligand.astype(jnp.float32)
    xm = jnp.concatenate(
        [x, mf[:, None], jnp.zeros((n, 12), jnp.float32)], axis=-1)

    ec = e // 2
    srcs = [src[:ec], src[ec:]]
    dsts = [dst[:ec], dst[ec:]]

    WdP1, bPd1, WsP1 = _proj_weights(params['hk'], params['hv'])
    tda1, tdb1, tsrc1 = _call_node1(
        h, xm, _mlp_weights(params['hq']) + [WdP1, bPd1, WsP1])
    ew1 = _prep_attn_weights(params['hk'], params['hv'], params['ew_h'])
    Sns, Sds = [], []
    for c in range(2):
        GdA = _sc_gather(tda1, dsts[c])
        GdB = _sc_gather(tdb1, dsts[c])
        Gs = _sc_gather(tsrc1, srcs[c])
        wv, ex = _call_edge(_edge1_body, GdA, GdB, Gs, ew1, [128, 128])
        (Sn,) = _sc_scatter_add([wv], dsts[c], n)
        (Sd,) = _sc_scatter_add([ex], dsts[c], n)
        Sns.append(Sn)
        Sds.append(Sd)

    pno = params['node_out']
    WdP2, bPd2, WsP2 = _proj_weights(params['xk'], params['xv'])
    ws2 = [pno['W1'][:HID], pno['W1'][HID:], _row(pno['b1']), _row(pno['g']),
           _row(pno['bln']), pno['W2'], _row(pno['b2'])] \
        + _mlp_weights(params['xq']) + [WdP2, bPd2, WsP2]
    h_new, tda2, tdb2, tsrc2 = _call_node2(
        Sns[0], Sns[1], Sds[0], Sds[1], h, xm, ws2)

    ew2 = _prep_attn_weights(params['xk'], params['xv'], params['ew_x'])
    Us = []
    for c in range(2):
        GdA = _sc_gather(tda2, dsts[c])
        GdB = _sc_gather(tdb2, dsts[c])
        Gs = _sc_gather(tsrc2, srcs[c])
        rows2 = _call_edge(_edge2_body, GdA, GdB, Gs, ew2, [128])
        (U,) = _sc_scatter_add([rows2], dsts[c], n)
        Us.append(U)
    xout = _call_node3(Us[0], Us[1], xm)
    return h_new, xout[:, :3]


# speedup vs baseline: 27.9513x; 1.1533x over previous
import functools

import jax
import jax.numpy as jnp
import numpy as np
from jax import lax
from jax.experimental import pallas as pl
from jax.experimental.pallas import tpu as pltpu
from jax.experimental.pallas import tpu_sc as plsc

NH = 16
HD = 8
HID = 128
NG = 20
RF = NG * 4
R_MAX = 10.0
_GSTEP = R_MAX / (NG - 1)
_GCOEFF = -0.5 / _GSTEP ** 2
_ISQ = 1.0 / float(np.sqrt(HD))

_TN = 1000
_TE = 640
_SCW = 128


def _ln_relu(y, g, b):
    mu = jnp.mean(y, -1, keepdims=True)
    var = jnp.mean((y - mu) ** 2, -1, keepdims=True)
    return jnp.maximum((y - mu) * lax.rsqrt(var + 1e-5) * g + b, 0.0)


def _mlp(xv, W1, b1, g, bln, W2, b2):
    y = jnp.dot(xv, W1, preferred_element_type=jnp.float32) + b1
    y = _ln_relu(y, g, bln)
    return jnp.dot(y, W2, preferred_element_type=jnp.float32) + b2


def _pack2(top, bot):
    t = lax.bitcast_convert_type(
        top.astype(jnp.bfloat16).astype(jnp.float32), jnp.int32)
    b = lax.bitcast_convert_type(
        bot.astype(jnp.bfloat16).astype(jnp.float32), jnp.int32)
    return jnp.bitwise_or(jnp.bitwise_and(t, jnp.int32(-65536)),
                          jnp.right_shift(jnp.bitwise_and(b, jnp.int32(-65536)), 16)
                          & jnp.int32(65535))


def _unpack_top(w):
    return lax.bitcast_convert_type(
        jnp.bitwise_and(w, jnp.int32(-65536)), jnp.float32)


def _unpack_bot(w):
    return lax.bitcast_convert_type(jnp.left_shift(w, 16), jnp.float32)


def _aux_vec(xm, width):
    x3 = xm[:, 0:3]
    xh = x3.astype(jnp.bfloat16).astype(jnp.float32)
    xl = x3 - xh
    pad = jnp.zeros((xm.shape[0], width - 7), jnp.float32)
    return jnp.concatenate([xh, xl, xm[:, 3:4], pad], axis=-1)


def _node1_body(h_ref, xm_ref, Wq1, bq1, gq, blq, Wq2, bq2,
                WdP, bPd, WsP, td_ref, ts_ref):
    h = h_ref[...]
    q1 = _mlp(h, Wq1[...], bq1[...], gq[...], blq[...], Wq2[...], bq2[...])
    Pd = jnp.dot(h, WdP[...], preferred_element_type=jnp.float32) + bPd[...]
    Ps = jnp.dot(h, WsP[...], preferred_element_type=jnp.float32)
    aux = _aux_vec(xm_ref[...], 128)
    td_ref[...] = _pack2(Pd, jnp.concatenate([q1, aux], axis=-1))
    ts_ref[...] = _pack2(Ps, jnp.concatenate([aux, jnp.zeros_like(Ps[:, :128])],
                                             axis=-1))


def _geom(xhi, xli, mi, xhj, xlj, mj):
    relx = (xhi[:, 0:1] + xli[:, 0:1]) - (xhj[:, 0:1] + xlj[:, 0:1])
    rely = (xhi[:, 1:2] + xli[:, 1:2]) - (xhj[:, 1:2] + xlj[:, 1:2])
    relz = (xhi[:, 2:3] + xli[:, 2:3]) - (xhj[:, 2:3] + xlj[:, 2:3])
    d = jnp.sqrt(relx * relx + rely * rely + relz * relz + 1e-12)
    offs = lax.broadcasted_iota(jnp.int32, (d.shape[0], NG), 1).astype(jnp.float32) * _GSTEP
    gauss = jnp.exp(_GCOEFF * (d - offs) ** 2)
    oh = jnp.concatenate(
        [(1.0 - mj) * (1.0 - mi), (1.0 - mj) * mi, mj * (1.0 - mi), mj * mi],
        axis=-1)
    return gauss, oh, jnp.concatenate([relx, rely, relz], axis=-1)


def _attn_core(gauss, oh, Pdk, Pdv, Psk, Psv, q,
               Wcat, gk, bk, W2k, b2k, gv, bv, W2v, b2v, ewb, HS):
    Gr = jnp.dot(gauss, Wcat, preferred_element_type=jnp.float32)
    oh_t = [oh[:, t:t + 1] for t in range(4)]
    racc_k = sum(oh_t[t] * Gr[:, t * 128:(t + 1) * 128] for t in range(4))
    racc_v = sum(oh_t[t] * Gr[:, 512 + t * 128:512 + (t + 1) * 128] for t in range(4))
    racc_e = sum(oh_t[t] * Gr[:, 1024 + t:1025 + t] for t in range(4))
    y_k = racc_k + Pdk + Psk
    y_v = racc_v + Pdv + Psv
    k_ = jnp.dot(_ln_relu(y_k, gk, bk), W2k, preferred_element_type=jnp.float32) + b2k
    v_ = jnp.dot(_ln_relu(y_v, gv, bv), W2v, preferred_element_type=jnp.float32) + b2v
    e_w = 1.0 / (1.0 + jnp.exp(-(racc_e + ewb)))
    logits = jnp.dot(q * k_, HS, preferred_element_type=jnp.float32) * _ISQ
    expl = jnp.exp(logits)
    return v_, e_w, expl


def _edge1_body(gd_ref, gs_ref, Wcat, gk, bk, W2k, b2k, gv, bv, W2v, b2v,
                ewb, HS, EX, wv_ref, ex_ref, geo_ref):
    dt = _unpack_top(gd_ref[...])
    db = _unpack_bot(gd_ref[...])
    st = _unpack_top(gs_ref[...])
    sb = _unpack_bot(gs_ref[...])
    gauss, oh, rel = _geom(db[:, 128:131], db[:, 131:134], db[:, 134:135],
                           sb[:, 0:3], sb[:, 3:6], sb[:, 6:7])
    v_, e_w, expl = _attn_core(
        gauss, oh, dt[:, 0:128], dt[:, 128:256], st[:, 0:128], st[:, 128:256],
        db[:, 0:128], Wcat[...], gk[...], bk[...], W2k[...], b2k[...],
        gv[...], bv[...], W2v[...], b2v[...], ewb[...], HS[...])
    expl_x = jnp.dot(expl, EX[...], preferred_element_type=jnp.float32)
    wv_ref[...] = (v_ * e_w) * expl_x
    ex_ref[...] = expl_x
    pad = jnp.zeros((gauss.shape[0], 5), jnp.float32)
    geo_ref[...] = jnp.concatenate([gauss, oh, rel, pad], axis=-1)


def _edge2_body(gd_ref, gs_ref, geo_ref, Wcat, gk, bk, W2k, b2k, gv, bv,
                W2v, b2v, ewb, HS, EX, rows_ref):
    dt = _unpack_top(gd_ref[...])
    db = _unpack_bot(gd_ref[...])
    geo = geo_ref[...]
    gauss = geo[:, 0:20]
    oh = geo[:, 20:24]
    v2, ew2, expl2 = _attn_core(
        gauss, oh, dt[:, 0:128], dt[:, 128:256],
        _unpack_top(gs_ref[...]), _unpack_bot(gs_ref[...]),
        db[:, 0:128], Wcat[...], gk[...], bk[...], W2k[...], b2k[...],
        gv[...], bv[...], W2v[...], b2v[...], ewb[...], HS[...])
    w2 = expl2 * (v2 * ew2)
    z = jnp.zeros((w2.shape[0], 64), jnp.float32)
    rows_ref[...] = jnp.concatenate(
        [w2 * geo[:, 24:25], w2 * geo[:, 25:26], w2 * geo[:, 26:27], expl2, z],
        axis=-1)


def _node2_body(Sn1_ref, Sn2_ref, Sd1_ref, Sd2_ref, h_ref, xm_ref,
                W1noA, W1noB, b1no, gno, blno, W2no, b2no,
                Wq1, bq1, gq, blq, Wq2, bq2, WdP, bPd, WsP,
                hnew_ref, td_ref, ts_ref):
    h = h_ref[...]
    num = Sn1_ref[0] + Sn1_ref[1] + Sn2_ref[0] + Sn2_ref[1]
    den = Sd1_ref[0] + Sd1_ref[1] + Sd2_ref[0] + Sd2_ref[1]
    out_attn = num / (den + 1e-16)
    y = (jnp.dot(out_attn, W1noA[...], preferred_element_type=jnp.float32)
         + jnp.dot(h, W1noB[...], preferred_element_type=jnp.float32) + b1no[...])
    out = jnp.dot(_ln_relu(y, gno[...], blno[...]), W2no[...],
                  preferred_element_type=jnp.float32) + b2no[...]
    h_new = out + h
    hnew_ref[...] = h_new
    q2 = _mlp(h_new, Wq1[...], bq1[...], gq[...], blq[...], Wq2[...], bq2[...])
    Pd = jnp.dot(h_new, WdP[...], preferred_element_type=jnp.float32) + bPd[...]
    Ps = jnp.dot(h_new, WsP[...], preferred_element_type=jnp.float32)
    z = jnp.zeros_like(q2)
    td_ref[...] = _pack2(Pd, jnp.concatenate([q2, z], axis=-1))
    ts_ref[...] = _pack2(Ps[:, 0:128], Ps[:, 128:256])


def _node3_body(U1_ref, U2_ref, xm_ref, xout_ref):
    U = U1_ref[0] + U1_ref[1] + U2_ref[0] + U2_ref[1]
    inv = 1.0 / (U[:, 48:64] + 1e-16)
    c = [jnp.mean(U[:, 16 * t:16 * t + 16] * inv, -1, keepdims=True)
         for t in range(3)]
    xm = xm_ref[...]
    mf = xm[:, 3:4]
    xnew = xm[:, 0:3] + jnp.concatenate(c, axis=-1) * mf
    pad = jnp.zeros((xnew.shape[0], 13), jnp.float32)
    xout_ref[...] = jnp.concatenate([xnew, pad], axis=-1)


def _full(a):
    return pl.BlockSpec(a.shape, lambda i: tuple(0 for _ in a.shape))


def _call_node1(h, xm, ws):
    n = h.shape[0]
    grid = (n // _TN,)
    ins = [pl.BlockSpec((_TN, 128), lambda i: (i, 0)),
           pl.BlockSpec((_TN, 16), lambda i: (i, 0))] + [_full(w) for w in ws]
    outs = [pl.BlockSpec((_TN, 256), lambda i: (i, 0)),
            pl.BlockSpec((_TN, 256), lambda i: (i, 0))]
    return pl.pallas_call(
        _node1_body, grid=grid, in_specs=ins, out_specs=outs,
        out_shape=[jax.ShapeDtypeStruct((n, 256), jnp.int32),
                   jax.ShapeDtypeStruct((n, 256), jnp.int32)],
    )(h, xm, *ws)


def _call_edge1(Gd, Gs, ws):
    e = Gd.shape[0]
    grid = (e // _TE,)
    ins = [pl.BlockSpec((_TE, 256), lambda i: (i, 0)),
           pl.BlockSpec((_TE, 256), lambda i: (i, 0))] + [_full(w) for w in ws]
    outs = [pl.BlockSpec((_TE, 128), lambda i: (i, 0)),
            pl.BlockSpec((_TE, 128), lambda i: (i, 0)),
            pl.BlockSpec((_TE, 32), lambda i: (i, 0))]
    return pl.pallas_call(
        _edge1_body, grid=grid, in_specs=ins, out_specs=outs,
        out_shape=[jax.ShapeDtypeStruct((e, 128), jnp.float32),
                   jax.ShapeDtypeStruct((e, 128), jnp.float32),
                   jax.ShapeDtypeStruct((e, 32), jnp.float32)],
    )(Gd, Gs, *ws)


def _call_edge2(Gd, Gs, geo, ws):
    e = Gd.shape[0]
    grid = (e // _TE,)
    ins = [pl.BlockSpec((_TE, 256), lambda i: (i, 0)),
           pl.BlockSpec((_TE, 128), lambda i: (i, 0)),
           pl.BlockSpec((_TE, 32), lambda i: (i, 0))] + [_full(w) for w in ws]
    outs = pl.BlockSpec((_TE, 128), lambda i: (i, 0))
    return pl.pallas_call(
        _edge2_body, grid=grid, in_specs=ins, out_specs=outs,
        out_shape=jax.ShapeDtypeStruct((e, 128), jnp.float32),
    )(Gd, Gs, geo, *ws)


def _call_node2(Sn1, Sn2, Sd1, Sd2, h, xm, ws):
    n = h.shape[0]
    grid = (n // _TN,)
    ins = [pl.BlockSpec((2, _TN, 128), lambda i: (0, i, 0)),
           pl.BlockSpec((2, _TN, 128), lambda i: (0, i, 0)),
           pl.BlockSpec((2, _TN, 128), lambda i: (0, i, 0)),
           pl.BlockSpec((2, _TN, 128), lambda i: (0, i, 0)),
           pl.BlockSpec((_TN, 128), lambda i: (i, 0)),
           pl.BlockSpec((_TN, 16), lambda i: (i, 0))] + [_full(w) for w in ws]
    outs = [pl.BlockSpec((_TN, 128), lambda i: (i, 0)),
            pl.BlockSpec((_TN, 256), lambda i: (i, 0)),
            pl.BlockSpec((_TN, 128), lambda i: (i, 0))]
    return pl.pallas_call(
        _node2_body, grid=grid, in_specs=ins, out_specs=outs,
        out_shape=[jax.ShapeDtypeStruct((n, 128), jnp.float32),
                   jax.ShapeDtypeStruct((n, 256), jnp.int32),
                   jax.ShapeDtypeStruct((n, 128), jnp.int32)],
    )(Sn1, Sn2, Sd1, Sd2, h, xm, *ws)


def _call_node3(U1, U2, xm):
    n = xm.shape[0]
    grid = (n // _TN,)
    ins = [pl.BlockSpec((2, _TN, 128), lambda i: (0, i, 0)),
           pl.BlockSpec((2, _TN, 128), lambda i: (0, i, 0)),
           pl.BlockSpec((_TN, 16), lambda i: (i, 0))]
    outs = pl.BlockSpec((_TN, 16), lambda i: (i, 0))
    return pl.pallas_call(
        _node3_body, grid=grid, in_specs=ins, out_specs=outs,
        out_shape=jax.ShapeDtypeStruct((n, 16), jnp.float32),
    )(U1, U2, xm)


def _sc_gather(table, idx):
    eg = idx.shape[0]
    d = table.shape[1]
    mesh = plsc.VectorSubcoreMesh(core_axis_name="core", subcore_axis_name="subcore")

    @functools.partial(
        pl.kernel,
        out_type=jax.ShapeDtypeStruct((eg, d), table.dtype),
        mesh=mesh)
    def k(t_hbm, i_hbm, o_hbm):
        def body(i_vmem, o_vmem):
            pltpu.sync_copy(t_hbm.at[i_vmem.at[0]], o_vmem)

        pltpu.emit_pipeline(
            body,
            grid=(eg // _SCW,),
            in_specs=[pl.BlockSpec((1, _SCW), lambda i: (0, i))],
            out_specs=[pl.BlockSpec((_SCW, d), lambda i: (i, 0))],
            core_axis_name=("core", "subcore"),
            dimension_semantics=(pltpu.PARALLEL,),
        )(i_hbm, o_hbm)

    return k(table, idx.reshape(1, eg))


def _sc_scatter_add(rows_list, idx, num_nodes):
    nrow = idx.shape[0] // _SCW
    dims = [r.shape[1] for r in rows_list]
    base, rem = nrow // 32, nrow % 32
    rpt = (num_nodes // 16) & ~7
    tail = num_nodes - 16 * rpt
    mesh = plsc.VectorSubcoreMesh(core_axis_name="core", subcore_axis_name="subcore")
    zeros = [jnp.zeros((num_nodes, d), jnp.float32) for d in dims]

    scratch = [pltpu.VMEM((_SCW,), jnp.int32)]
    for d in dims:
        scratch.append(pltpu.VMEM((_SCW, d), jnp.float32))
        scratch.append(pltpu.VMEM_SHARED((num_nodes, d), jnp.float32))

    @functools.partial(
        pl.kernel,
        out_type=[jax.ShapeDtypeStruct((2, num_nodes, d), jnp.float32)
                  for d in dims],
        mesh=mesh,
        scratch_types=scratch)
    def k(*refs):
        nz = len(dims)
        z_hbm = refs[0:nz]
        r_hbm = refs[nz:2 * nz]
        i_hbm = refs[2 * nz]
        o_hbm = refs[2 * nz + 1:3 * nz + 1]
        idxb = refs[3 * nz + 1]
        bufs = refs[3 * nz + 2::2]
        shareds = refs[3 * nz + 3::2]

        cid = lax.axis_index("core")
        sid = lax.axis_index("subcore")
        wid = sid * 2 + cid

        @pl.when(sid == 0)
        def _():
            for z, sh in zip(z_hbm, shareds):
                pltpu.sync_copy(z, sh)

        plsc.subcore_barrier()

        start = wid * base + jnp.minimum(wid, rem)
        cnt = base + jnp.where(wid < rem, 1, 0)

        @pl.loop(0, cnt)
        def _(t):
            off = pl.multiple_of((start + t) * _SCW, _SCW)
            pltpu.sync_copy(i_hbm.at[pl.ds(off, _SCW)], idxb)
            for r, buf, sh in zip(r_hbm, bufs, shareds):
                pltpu.sync_copy(r.at[pl.ds(off, _SCW)], buf)
                pltpu.sync_copy(buf, sh.at[idxb], add=True)

        plsc.subcore_barrier()
        ns = pl.multiple_of(sid * rpt, 8)
        for sh, o in zip(shareds, o_hbm):
            pltpu.sync_copy(sh.at[pl.ds(ns, rpt)], o.at[cid, pl.ds(ns, rpt)])
        if tail:
            @pl.when(sid == 0)
            def _():
                for sh, o in zip(shareds, o_hbm):
                    pltpu.sync_copy(sh.at[pl.ds(16 * rpt, tail)],
                                    o.at[cid, pl.ds(16 * rpt, tail)])

    return k(*zeros, *rows_list, idx)


def _r_weight(W1):
    return jnp.concatenate([W1[t:RF:4, :] for t in range(4)], axis=1)


def _row(v):
    return v.reshape(1, -1)


def _prep_attn_weights(pk, pv, pew):
    Wcat = jnp.concatenate(
        [_r_weight(pk['W1']), _r_weight(pv['W1']),
         jnp.concatenate([pew['W'][t:RF:4, :] for t in range(4)], axis=1)],
        axis=1)
    HS = (jnp.arange(128)[:, None] // HD
          == jnp.arange(NH)[None, :]).astype(jnp.float32)
    EX = HS.T
    return [Wcat, _row(pk['g']), _row(pk['bln']), pk['W2'], _row(pk['b2']),
            _row(pv['g']), _row(pv['bln']), pv['W2'], _row(pv['b2']),
            pew['b'].reshape(1, 1), HS, EX]


def _proj_weights(pk, pv):
    WdP = jnp.concatenate([pk['W1'][RF:RF + HID], pv['W1'][RF:RF + HID]], axis=1)
    bPd = _row(jnp.concatenate([pk['b1'], pv['b1']]))
    WsP = jnp.concatenate([pk['W1'][RF + HID:], pv['W1'][RF + HID:]], axis=1)
    return WdP, bPd, WsP


def _mlp_weights(p):
    return [p['W1'], _row(p['b1']), _row(p['g']), _row(p['bln']),
            p['W2'], _row(p['b2'])]


def kernel(h, x, params, edge_index, mask_ligand):
    n = h.shape[0]
    e = edge_index.shape[1]
    src = edge_index[0].astype(jnp.int32)
    dst = edge_index[1].astype(jnp.int32)
    mf = mask_ligand.astype(jnp.float32)
    xm = jnp.concatenate(
        [x, mf[:, None], jnp.zeros((n, 12), jnp.float32)], axis=-1)
    ec = e // 2
    srcs = [src[:ec], src[ec:]]
    dsts = [dst[:ec], dst[ec:]]

    WdP1, bPd1, WsP1 = _proj_weights(params['hk'], params['hv'])
    td1, ts1 = _call_node1(
        h, xm, _mlp_weights(params['hq']) + [WdP1, bPd1, WsP1])
    ew1 = _prep_attn_weights(params['hk'], params['hv'], params['ew_h'])
    Sns, Sds, geos = [], [], []
    for c in range(2):
        Gd = _sc_gather(td1, dsts[c])
        Gs = _sc_gather(ts1, srcs[c])
        wv, ex, geo = _call_edge1(Gd, Gs, ew1)
        geos.append(geo)
        (Sn,) = _sc_scatter_add([wv], dsts[c], n)
        (Sd,) = _sc_scatter_add([ex], dsts[c], n)
        Sns.append(Sn)
        Sds.append(Sd)

    pno = params['node_out']
    WdP2, bPd2, WsP2 = _proj_weights(params['xk'], params['xv'])
    ws2 = [pno['W1'][:HID], pno['W1'][HID:], _row(pno['b1']), _row(pno['g']),
           _row(pno['bln']), pno['W2'], _row(pno['b2'])] \
        + _mlp_weights(params['xq']) + [WdP2, bPd2, WsP2]
    h_new, td2, ts2 = _call_node2(Sns[0], Sns[1], Sds[0], Sds[1], h, xm, ws2)

    ew2 = _prep_attn_weights(params['xk'], params['xv'], params['ew_x'])
    Us = []
    for c in range(2):
        Gd = _sc_gather(td2, dsts[c])
        Gs = _sc_gather(ts2, srcs[c])
        rows2 = _call_edge2(Gd, Gs, geos[c], ew2)
        (U,) = _sc_scatter_add([rows2], dsts[c], n)
        Us.append(U)
    xout = _call_node3(Us[0], Us[1], xm)
    return h_new, xout[:, :3]


# speedup vs baseline: 29.3135x; 1.0487x over previous
import functools

import jax
import jax.numpy as jnp
import numpy as np
from jax import lax
from jax.experimental import pallas as pl
from jax.experimental.pallas import tpu as pltpu
from jax.experimental.pallas import tpu_sc as plsc

NH = 16
HD = 8
HID = 128
NG = 20
RF = NG * 4
R_MAX = 10.0
_GSTEP = R_MAX / (NG - 1)
_GCOEFF = -0.5 / _GSTEP ** 2
_ISQ = 1.0 / float(np.sqrt(HD))

_TN = 1000
_TE = 640
_SCW = 128


def _ln_relu(y, g, b):
    mu = jnp.mean(y, -1, keepdims=True)
    var = jnp.mean((y - mu) ** 2, -1, keepdims=True)
    return jnp.maximum((y - mu) * lax.rsqrt(var + 1e-5) * g + b, 0.0)


def _mlp(xv, W1, b1, g, bln, W2, b2):
    y = jnp.dot(xv, W1, preferred_element_type=jnp.float32) + b1
    y = _ln_relu(y, g, bln)
    return jnp.dot(y, W2, preferred_element_type=jnp.float32) + b2


def _pack2(top, bot):
    t = lax.bitcast_convert_type(
        top.astype(jnp.bfloat16).astype(jnp.float32), jnp.int32)
    b = lax.bitcast_convert_type(
        bot.astype(jnp.bfloat16).astype(jnp.float32), jnp.int32)
    return jnp.bitwise_or(jnp.bitwise_and(t, jnp.int32(-65536)),
                          jnp.right_shift(jnp.bitwise_and(b, jnp.int32(-65536)), 16)
                          & jnp.int32(65535))


def _unpack_top(w):
    return lax.bitcast_convert_type(
        jnp.bitwise_and(w, jnp.int32(-65536)), jnp.float32)


def _unpack_bot(w):
    return lax.bitcast_convert_type(jnp.left_shift(w, 16), jnp.float32)


def _aux_vec(xm, width):
    x3 = xm[:, 0:3]
    xh = x3.astype(jnp.bfloat16).astype(jnp.float32)
    xl = x3 - xh
    pad = jnp.zeros((xm.shape[0], width - 7), jnp.float32)
    return jnp.concatenate([xh, xl, xm[:, 3:4], pad], axis=-1)


def _node1_body(h_ref, xm_ref, Wq1, bq1, gq, blq, Wq2, bq2,
                WdP, bPd, WsP, td_ref, ts_ref):
    h = h_ref[...]
    q1 = _mlp(h, Wq1[...], bq1[...], gq[...], blq[...], Wq2[...], bq2[...])
    Pd = jnp.dot(h, WdP[...], preferred_element_type=jnp.float32) + bPd[...]
    Ps = jnp.dot(h, WsP[...], preferred_element_type=jnp.float32)
    aux = _aux_vec(xm_ref[...], 128)
    td_ref[...] = _pack2(Pd, jnp.concatenate([q1, aux], axis=-1))
    ts_ref[...] = _pack2(Ps, jnp.concatenate([aux, jnp.zeros_like(Ps[:, :128])],
                                             axis=-1))


def _geom(xhi, xli, mi, xhj, xlj, mj):
    relx = (xhi[:, 0:1] + xli[:, 0:1]) - (xhj[:, 0:1] + xlj[:, 0:1])
    rely = (xhi[:, 1:2] + xli[:, 1:2]) - (xhj[:, 1:2] + xlj[:, 1:2])
    relz = (xhi[:, 2:3] + xli[:, 2:3]) - (xhj[:, 2:3] + xlj[:, 2:3])
    d = jnp.sqrt(relx * relx + rely * rely + relz * relz + 1e-12)
    offs = lax.broadcasted_iota(jnp.int32, (d.shape[0], NG), 1).astype(jnp.float32) * _GSTEP
    gauss = jnp.exp(_GCOEFF * (d - offs) ** 2)
    oh = jnp.concatenate(
        [(1.0 - mj) * (1.0 - mi), (1.0 - mj) * mi, mj * (1.0 - mi), mj * mi],
        axis=-1)
    return gauss, oh, jnp.concatenate([relx, rely, relz], axis=-1)


def _attn_core(gauss, oh, Pdk, Pdv, Psk, Psv, q,
               Wcat, gk, bk, W2k, b2k, gv, bv, W2v, b2v, ewb, HS):
    Gr = jnp.dot(gauss, Wcat, preferred_element_type=jnp.float32)
    oh_t = [oh[:, t:t + 1] for t in range(4)]
    racc_k = sum(oh_t[t] * Gr[:, t * 128:(t + 1) * 128] for t in range(4))
    racc_v = sum(oh_t[t] * Gr[:, 512 + t * 128:512 + (t + 1) * 128] for t in range(4))
    racc_e = sum(oh_t[t] * Gr[:, 1024 + t:1025 + t] for t in range(4))
    y_k = racc_k + Pdk + Psk
    y_v = racc_v + Pdv + Psv
    k_ = jnp.dot(_ln_relu(y_k, gk, bk), W2k, preferred_element_type=jnp.float32) + b2k
    v_ = jnp.dot(_ln_relu(y_v, gv, bv), W2v, preferred_element_type=jnp.float32) + b2v
    e_w = 1.0 / (1.0 + jnp.exp(-(racc_e + ewb)))
    logits = jnp.dot(q * k_, HS, preferred_element_type=jnp.float32) * _ISQ
    expl = jnp.exp(logits)
    return v_, e_w, expl


def _edge1_body(gd_ref, gs_ref, Wcat, gk, bk, W2k, b2k, gv, bv, W2v, b2v,
                ewb, HS, EX, wv_ref, ex_ref, geo_ref):
    dt = _unpack_top(gd_ref[...])
    db = _unpack_bot(gd_ref[...])
    st = _unpack_top(gs_ref[...])
    sb = _unpack_bot(gs_ref[...])
    gauss, oh, rel = _geom(db[:, 128:131], db[:, 131:134], db[:, 134:135],
                           sb[:, 0:3], sb[:, 3:6], sb[:, 6:7])
    v_, e_w, expl = _attn_core(
        gauss, oh, dt[:, 0:128], dt[:, 128:256], st[:, 0:128], st[:, 128:256],
        db[:, 0:128], Wcat[...], gk[...], bk[...], W2k[...], b2k[...],
        gv[...], bv[...], W2v[...], b2v[...], ewb[...], HS[...])
    expl_x = jnp.dot(expl, EX[...], preferred_element_type=jnp.float32)
    wv_ref[...] = (v_ * e_w) * expl_x
    ex_ref[...] = expl_x
    pad = jnp.zeros((gauss.shape[0], 5), jnp.float32)
    geo_ref[...] = jnp.concatenate([gauss, oh, rel, pad], axis=-1)


def _edge2_body(gd_ref, gs_ref, geo_ref, Wcat, gk, bk, W2k, b2k, gv, bv,
                W2v, b2v, ewb, HS, EX, rows_ref):
    dt = _unpack_top(gd_ref[...])
    db = _unpack_bot(gd_ref[...])
    geo = geo_ref[...]
    gauss = geo[:, 0:20]
    oh = geo[:, 20:24]
    v2, ew2, expl2 = _attn_core(
        gauss, oh, dt[:, 0:128], dt[:, 128:256],
        _unpack_top(gs_ref[...]), _unpack_bot(gs_ref[...]),
        db[:, 0:128], Wcat[...], gk[...], bk[...], W2k[...], b2k[...],
        gv[...], bv[...], W2v[...], b2v[...], ewb[...], HS[...])
    w2 = expl2 * (v2 * ew2)
    z = jnp.zeros((w2.shape[0], 64), jnp.float32)
    rows_ref[...] = jnp.concatenate(
        [w2 * geo[:, 24:25], w2 * geo[:, 25:26], w2 * geo[:, 26:27], expl2, z],
        axis=-1)


def _node2_body(Sn1_ref, Sn2_ref, Sd1_ref, Sd2_ref, h_ref, xm_ref,
                W1noA, W1noB, b1no, gno, blno, W2no, b2no,
                Wq1, bq1, gq, blq, Wq2, bq2, WdP, bPd, WsP,
                hnew_ref, td_ref, ts_ref):
    h = h_ref[...]
    num = Sn1_ref[0] + Sn1_ref[1] + Sn2_ref[0] + Sn2_ref[1]
    den = Sd1_ref[0] + Sd1_ref[1] + Sd2_ref[0] + Sd2_ref[1]
    out_attn = num / (den + 1e-16)
    y = (jnp.dot(out_attn, W1noA[...], preferred_element_type=jnp.float32)
         + jnp.dot(h, W1noB[...], preferred_element_type=jnp.float32) + b1no[...])
    out = jnp.dot(_ln_relu(y, gno[...], blno[...]), W2no[...],
                  preferred_element_type=jnp.float32) + b2no[...]
    h_new = out + h
    hnew_ref[...] = h_new
    q2 = _mlp(h_new, Wq1[...], bq1[...], gq[...], blq[...], Wq2[...], bq2[...])
    Pd = jnp.dot(h_new, WdP[...], preferred_element_type=jnp.float32) + bPd[...]
    Ps = jnp.dot(h_new, WsP[...], preferred_element_type=jnp.float32)
    z = jnp.zeros_like(q2)
    td_ref[...] = _pack2(Pd, jnp.concatenate([q2, z], axis=-1))
    ts_ref[...] = _pack2(Ps[:, 0:128], Ps[:, 128:256])


def _node3_body(U1_ref, U2_ref, xm_ref, xout_ref):
    U = U1_ref[0] + U1_ref[1] + U2_ref[0] + U2_ref[1]
    inv = 1.0 / (U[:, 48:64] + 1e-16)
    c = [jnp.mean(U[:, 16 * t:16 * t + 16] * inv, -1, keepdims=True)
         for t in range(3)]
    xm = xm_ref[...]
    mf = xm[:, 3:4]
    xnew = xm[:, 0:3] + jnp.concatenate(c, axis=-1) * mf
    pad = jnp.zeros((xnew.shape[0], 13), jnp.float32)
    xout_ref[...] = jnp.concatenate([xnew, pad], axis=-1)


def _full(a):
    return pl.BlockSpec(a.shape, lambda i: tuple(0 for _ in a.shape))


def _call_node1(h, xm, ws):
    n = h.shape[0]
    grid = (n // _TN,)
    ins = [pl.BlockSpec((_TN, 128), lambda i: (i, 0)),
           pl.BlockSpec((_TN, 16), lambda i: (i, 0))] + [_full(w) for w in ws]
    outs = [pl.BlockSpec((_TN, 256), lambda i: (i, 0)),
            pl.BlockSpec((_TN, 256), lambda i: (i, 0))]
    return pl.pallas_call(
        _node1_body, grid=grid, in_specs=ins, out_specs=outs,
        out_shape=[jax.ShapeDtypeStruct((n, 256), jnp.int32),
                   jax.ShapeDtypeStruct((n, 256), jnp.int32)],
    )(h, xm, *ws)


def _call_edge1(Gd, Gs, ws):
    e = Gd.shape[0]
    grid = (e // _TE,)
    ins = [pl.BlockSpec((_TE, 256), lambda i: (i, 0)),
           pl.BlockSpec((_TE, 256), lambda i: (i, 0))] + [_full(w) for w in ws]
    outs = [pl.BlockSpec((_TE, 128), lambda i: (i, 0)),
            pl.BlockSpec((_TE, 128), lambda i: (i, 0)),
            pl.BlockSpec((_TE, 32), lambda i: (i, 0))]
    return pl.pallas_call(
        _edge1_body, grid=grid, in_specs=ins, out_specs=outs,
        out_shape=[jax.ShapeDtypeStruct((e, 128), jnp.float32),
                   jax.ShapeDtypeStruct((e, 128), jnp.float32),
                   jax.ShapeDtypeStruct((e, 32), jnp.float32)],
    )(Gd, Gs, *ws)


def _call_edge2(Gd, Gs, geo, ws):
    e = Gd.shape[0]
    grid = (e // _TE,)
    ins = [pl.BlockSpec((_TE, 256), lambda i: (i, 0)),
           pl.BlockSpec((_TE, 128), lambda i: (i, 0)),
           pl.BlockSpec((_TE, 32), lambda i: (i, 0))] + [_full(w) for w in ws]
    outs = pl.BlockSpec((_TE, 128), lambda i: (i, 0))
    return pl.pallas_call(
        _edge2_body, grid=grid, in_specs=ins, out_specs=outs,
        out_shape=jax.ShapeDtypeStruct((e, 128), jnp.float32),
    )(Gd, Gs, geo, *ws)


def _call_node2(Sn1, Sn2, Sd1, Sd2, h, xm, ws):
    n = h.shape[0]
    grid = (n // _TN,)
    ins = [pl.BlockSpec((2, _TN, 128), lambda i: (0, i, 0)),
           pl.BlockSpec((2, _TN, 128), lambda i: (0, i, 0)),
           pl.BlockSpec((2, _TN, 128), lambda i: (0, i, 0)),
           pl.BlockSpec((2, _TN, 128), lambda i: (0, i, 0)),
           pl.BlockSpec((_TN, 128), lambda i: (i, 0)),
           pl.BlockSpec((_TN, 16), lambda i: (i, 0))] + [_full(w) for w in ws]
    outs = [pl.BlockSpec((_TN, 128), lambda i: (i, 0)),
            pl.BlockSpec((_TN, 256), lambda i: (i, 0)),
            pl.BlockSpec((_TN, 128), lambda i: (i, 0))]
    return pl.pallas_call(
        _node2_body, grid=grid, in_specs=ins, out_specs=outs,
        out_shape=[jax.ShapeDtypeStruct((n, 128), jnp.float32),
                   jax.ShapeDtypeStruct((n, 256), jnp.int32),
                   jax.ShapeDtypeStruct((n, 128), jnp.int32)],
    )(Sn1, Sn2, Sd1, Sd2, h, xm, *ws)


def _call_node3(U1, U2, xm):
    n = xm.shape[0]
    grid = (n // _TN,)
    ins = [pl.BlockSpec((2, _TN, 128), lambda i: (0, i, 0)),
           pl.BlockSpec((2, _TN, 128), lambda i: (0, i, 0)),
           pl.BlockSpec((_TN, 16), lambda i: (i, 0))]
    outs = pl.BlockSpec((_TN, 16), lambda i: (i, 0))
    return pl.pallas_call(
        _node3_body, grid=grid, in_specs=ins, out_specs=outs,
        out_shape=jax.ShapeDtypeStruct((n, 16), jnp.float32),
    )(U1, U2, xm)


def _sc_gather(table, idx):
    eg = idx.shape[0]
    d = table.shape[1]
    mesh = plsc.VectorSubcoreMesh(core_axis_name="core", subcore_axis_name="subcore")

    @functools.partial(
        pl.kernel,
        out_type=jax.ShapeDtypeStruct((eg, d), table.dtype),
        mesh=mesh)
    def k(t_hbm, i_hbm, o_hbm):
        def body(i_vmem, o_vmem):
            pltpu.sync_copy(t_hbm.at[i_vmem.at[0]], o_vmem)

        pltpu.emit_pipeline(
            body,
            grid=(eg // _SCW,),
            in_specs=[pl.BlockSpec((1, _SCW), lambda i: (0, i))],
            out_specs=[pl.BlockSpec((_SCW, d), lambda i: (i, 0))],
            core_axis_name=("core", "subcore"),
            dimension_semantics=(pltpu.PARALLEL,),
        )(i_hbm, o_hbm)

    return k(table, idx.reshape(1, eg))


def _sc_scatter_add(rows_list, idx, num_nodes):
    nrow = idx.shape[0] // _SCW
    assert len(rows_list) == 1
    d = rows_list[0].shape[1]
    base, rem = nrow // 32, nrow % 32
    rpt = (num_nodes // 16) & ~7
    tail = num_nodes - 16 * rpt
    mesh = plsc.VectorSubcoreMesh(core_axis_name="core", subcore_axis_name="subcore")
    zeros = jnp.zeros((num_nodes, d), jnp.float32)

    scratch = [pltpu.VMEM((2, _SCW), jnp.int32),
               pltpu.VMEM((2, _SCW, d), jnp.float32),
               pltpu.VMEM_SHARED((num_nodes, d), jnp.float32),
               pltpu.SemaphoreType.DMA((2,)),
               pltpu.SemaphoreType.DMA((2,))]

    @functools.partial(
        pl.kernel,
        out_type=[jax.ShapeDtypeStruct((2, num_nodes, d), jnp.float32)],
        mesh=mesh,
        scratch_types=scratch)
    def k(z_hbm, r_hbm, i_hbm, o_hbm, idxb, rowb, sh, isem, rsem):
        cid = lax.axis_index("core")
        sid = lax.axis_index("subcore")
        wid = sid * 2 + cid

        @pl.when(sid == 0)
        def _():
            pltpu.sync_copy(z_hbm, sh)

        plsc.subcore_barrier()

        start = wid * base + jnp.minimum(wid, rem)
        cnt = base + jnp.where(wid < rem, 1, 0)

        def fetch(b, j):
            off = pl.multiple_of((start + j) * _SCW, _SCW)
            pltpu.async_copy(i_hbm.at[pl.ds(off, _SCW)], idxb.at[b], isem.at[b])
            pltpu.async_copy(r_hbm.at[pl.ds(off, _SCW)], rowb.at[b], rsem.at[b])

        def drain(b):
            pltpu.make_async_copy(i_hbm.at[pl.ds(0, _SCW)], idxb.at[b],
                                  isem.at[b]).wait()
            pltpu.make_async_copy(r_hbm.at[pl.ds(0, _SCW)], rowb.at[b],
                                  rsem.at[b]).wait()

        for b in range(2):
            @pl.when(cnt > b)
            def _(b=b):
                fetch(b, b)

        @pl.loop(0, cnt, step=2)
        def _(t):
            for b in range(2):
                @pl.when(t + b < cnt)
                def _(b=b):
                    drain(b)
                    pltpu.sync_copy(rowb.at[b], sh.at[idxb.at[b]], add=True)

                    @pl.when(t + b + 2 < cnt)
                    def _(b=b):
                        fetch(b, t + b + 2)

        plsc.subcore_barrier()
        ns = pl.multiple_of(sid * rpt, 8)
        pltpu.sync_copy(sh.at[pl.ds(ns, rpt)], o_hbm.at[cid, pl.ds(ns, rpt)])
        if tail:
            @pl.when(sid == 0)
            def _():
                pltpu.sync_copy(sh.at[pl.ds(16 * rpt, tail)],
                                o_hbm.at[cid, pl.ds(16 * rpt, tail)])

    return k(zeros, rows_list[0], idx)


def _r_weight(W1):
    return jnp.concatenate([W1[t:RF:4, :] for t in range(4)], axis=1)


def _row(v):
    return v.reshape(1, -1)


def _prep_attn_weights(pk, pv, pew):
    Wcat = jnp.concatenate(
        [_r_weight(pk['W1']), _r_weight(pv['W1']),
         jnp.concatenate([pew['W'][t:RF:4, :] for t in range(4)], axis=1)],
        axis=1)
    HS = (jnp.arange(128)[:, None] // HD
          == jnp.arange(NH)[None, :]).astype(jnp.float32)
    EX = HS.T
    return [Wcat, _row(pk['g']), _row(pk['bln']), pk['W2'], _row(pk['b2']),
            _row(pv['g']), _row(pv['bln']), pv['W2'], _row(pv['b2']),
            pew['b'].reshape(1, 1), HS, EX]


def _proj_weights(pk, pv):
    WdP = jnp.concatenate([pk['W1'][RF:RF + HID], pv['W1'][RF:RF + HID]], axis=1)
    bPd = _row(jnp.concatenate([pk['b1'], pv['b1']]))
    WsP = jnp.concatenate([pk['W1'][RF + HID:], pv['W1'][RF + HID:]], axis=1)
    return WdP, bPd, WsP


def _mlp_weights(p):
    return [p['W1'], _row(p['b1']), _row(p['g']), _row(p['bln']),
            p['W2'], _row(p['b2'])]


def kernel(h, x, params, edge_index, mask_ligand):
    n = h.shape[0]
    e = edge_index.shape[1]
    src = edge_index[0].astype(jnp.int32)
    dst = edge_index[1].astype(jnp.int32)
    mf = mask_ligand.astype(jnp.float32)
    xm = jnp.concatenate(
        [x, mf[:, None], jnp.zeros((n, 12), jnp.float32)], axis=-1)
    ec = e // 2
    srcs = [src[:ec], src[ec:]]
    dsts = [dst[:ec], dst[ec:]]

    WdP1, bPd1, WsP1 = _proj_weights(params['hk'], params['hv'])
    td1, ts1 = _call_node1(
        h, xm, _mlp_weights(params['hq']) + [WdP1, bPd1, WsP1])
    ew1 = _prep_attn_weights(params['hk'], params['hv'], params['ew_h'])
    Sns, Sds, geos = [], [], []
    for c in range(2):
        Gd = _sc_gather(td1, dsts[c])
        Gs = _sc_gather(ts1, srcs[c])
        wv, ex, geo = _call_edge1(Gd, Gs, ew1)
        geos.append(geo)
        (Sn,) = _sc_scatter_add([wv], dsts[c], n)
        (Sd,) = _sc_scatter_add([ex], dsts[c], n)
        Sns.append(Sn)
        Sds.append(Sd)

    pno = params['node_out']
    WdP2, bPd2, WsP2 = _proj_weights(params['xk'], params['xv'])
    ws2 = [pno['W1'][:HID], pno['W1'][HID:], _row(pno['b1']), _row(pno['g']),
           _row(pno['bln']), pno['W2'], _row(pno['b2'])] \
        + _mlp_weights(params['xq']) + [WdP2, bPd2, WsP2]
    h_new, td2, ts2 = _call_node2(Sns[0], Sns[1], Sds[0], Sds[1], h, xm, ws2)

    ew2 = _prep_attn_weights(params['xk'], params['xv'], params['ew_x'])
    Us = []
    for c in range(2):
        Gd = _sc_gather(td2, dsts[c])
        Gs = _sc_gather(ts2, srcs[c])
        rows2 = _call_edge2(Gd, Gs, geos[c], ew2)
        (U,) = _sc_scatter_add([rows2], dsts[c], n)
        Us.append(U)
    xout = _call_node3(Us[0], Us[1], xm)
    return h_new, xout[:, :3]


# speedup vs baseline: 35.7706x; 1.2203x over previous
import functools

import jax
import jax.numpy as jnp
import numpy as np
from jax import lax
from jax.experimental import pallas as pl
from jax.experimental.pallas import tpu as pltpu
from jax.experimental.pallas import tpu_sc as plsc

NH = 16
HD = 8
HID = 128
NG = 20
RF = NG * 4
R_MAX = 10.0
_GSTEP = R_MAX / (NG - 1)
_GCOEFF = -0.5 / _GSTEP ** 2
_ISQ = 1.0 / float(np.sqrt(HD))

_TN = 1000
_TE = 640
_SCW = 128


def _ln_relu(y, g, b):
    mu = jnp.mean(y, -1, keepdims=True)
    var = jnp.mean((y - mu) ** 2, -1, keepdims=True)
    return jnp.maximum((y - mu) * lax.rsqrt(var + 1e-5) * g + b, 0.0)


def _mlp(xv, W1, b1, g, bln, W2, b2):
    y = jnp.dot(xv, W1, preferred_element_type=jnp.float32) + b1
    y = _ln_relu(y, g, bln)
    return jnp.dot(y, W2, preferred_element_type=jnp.float32) + b2


def _pack2(top, bot):
    t = lax.bitcast_convert_type(
        top.astype(jnp.bfloat16).astype(jnp.float32), jnp.int32)
    b = lax.bitcast_convert_type(
        bot.astype(jnp.bfloat16).astype(jnp.float32), jnp.int32)
    return jnp.bitwise_or(jnp.bitwise_and(t, jnp.int32(-65536)),
                          jnp.right_shift(jnp.bitwise_and(b, jnp.int32(-65536)), 16)
                          & jnp.int32(65535))


def _unpack_top(w):
    return lax.bitcast_convert_type(w, jnp.float32)


def _unpack_bot(w):
    return lax.bitcast_convert_type(jnp.left_shift(w, 16), jnp.float32)


def _aux_vec(xm, width):
    x3 = xm[:, 0:3]
    xh = x3.astype(jnp.bfloat16).astype(jnp.float32)
    xl = x3 - xh
    pad = jnp.zeros((xm.shape[0], width - 7), jnp.float32)
    return jnp.concatenate([xh, xl, xm[:, 3:4], pad], axis=-1)


def _node1_body(h_ref, xm_ref, Wq1, bq1, gq, blq, Wq2, bq2,
                WdP, bPd, WsP, td_ref, ts_ref):
    h = h_ref[...]
    q1 = _mlp(h, Wq1[...], bq1[...], gq[...], blq[...], Wq2[...], bq2[...])
    Pd = jnp.dot(h, WdP[...], preferred_element_type=jnp.float32) + bPd[...]
    Ps = jnp.dot(h, WsP[...], preferred_element_type=jnp.float32)
    aux = _aux_vec(xm_ref[...], 128)
    td_ref[...] = _pack2(Pd, jnp.concatenate([q1, aux], axis=-1))
    ts_ref[...] = _pack2(Ps, jnp.concatenate([aux, jnp.zeros_like(Ps[:, :128])],
                                             axis=-1))


def _geom(db, sb):
    xi = db[:, 128:131] + db[:, 131:134]
    xj = sb[:, 0:3] + sb[:, 3:6]
    rel = xi - xj
    mi = db[:, 134:135]
    mj = sb[:, 6:7]
    d2 = jnp.dot(rel * rel, jnp.ones((3, 1), jnp.float32),
                 preferred_element_type=jnp.float32)
    dt = jnp.transpose(jnp.sqrt(d2 + 1e-12))
    offs = lax.broadcasted_iota(jnp.int32, (NG, 1), 0).astype(jnp.float32) * _GSTEP
    gauss_t = jnp.exp(_GCOEFF * (dt - offs) ** 2)
    oh = jnp.concatenate(
        [(1.0 - mj) * (1.0 - mi), (1.0 - mj) * mi, mj * (1.0 - mi), mj * mi],
        axis=-1)
    return gauss_t, oh, rel


def _attn_core(gauss_t, oh, Pdk, Pdv, Psk, Psv, q,
               Wr2, R, T, gk, bk, W2k, b2k, gv, bv, W2v, b2v, ewb, HS):
    r_feat = (lax.dot_general(gauss_t, R, (((0,), (0,)), ((), ())),
                              preferred_element_type=jnp.float32)
              * jnp.dot(oh, T, preferred_element_type=jnp.float32))
    Gr = jnp.dot(r_feat, Wr2, preferred_element_type=jnp.float32)
    y_k = Gr[:, 0:128] + Pdk + Psk
    y_v = Gr[:, 128:256] + Pdv + Psv
    k_ = jnp.dot(_ln_relu(y_k, gk, bk), W2k, preferred_element_type=jnp.float32) + b2k
    v_ = jnp.dot(_ln_relu(y_v, gv, bv), W2v, preferred_element_type=jnp.float32) + b2v
    logits = jnp.dot(q * k_, HS, preferred_element_type=jnp.float32) * _ISQ
    z = jnp.concatenate(
        [logits, -(Gr[:, 256:257] + ewb),
         jnp.zeros((logits.shape[0], 15), jnp.float32)], axis=-1)
    ez = jnp.exp(z)
    expl = ez[:, 0:16]
    e_w = 1.0 / (1.0 + ez[:, 16:17])
    return v_, e_w, expl


def _edge1_body(gd_ref, gs_ref, Wr2, R, T, gk, bk, W2k, b2k, gv, bv, W2v, b2v,
                ewb, HS, EX, wv_ref, ex_ref, geo8_ref, geot_ref):
    dt = _unpack_top(gd_ref[...])
    db = _unpack_bot(gd_ref[...])
    st = _unpack_top(gs_ref[...])
    sb = _unpack_bot(gs_ref[...])
    gauss_t, oh, rel = _geom(db, sb)
    v_, e_w, expl = _attn_core(
        gauss_t, oh, dt[:, 0:128], dt[:, 128:256], st[:, 0:128], st[:, 128:256],
        db[:, 0:128], Wr2[...], R[...], T[...], gk[...], bk[...], W2k[...],
        b2k[...], gv[...], bv[...], W2v[...], b2v[...], ewb[...], HS[...])
    expl_x = jnp.dot(expl, EX[...], preferred_element_type=jnp.float32)
    wv_ref[...] = (v_ * e_w) * expl_x
    ex_ref[...] = expl_x
    pad = jnp.zeros((oh.shape[0], 1), jnp.float32)
    geo8_ref[...] = jnp.concatenate([oh, rel, pad], axis=-1)
    geot_ref[...] = gauss_t


def _edge2_body(gd_ref, gs_ref, geo8_ref, geot_ref, Wr2, R, T, T4, M4, E4,
                gk, bk, W2k, b2k, gv, bv, W2v, b2v, ewb, HS, EX, rows_ref):
    dt = _unpack_top(gd_ref[...])
    db = _unpack_bot(gd_ref[...])
    geo8 = geo8_ref[...]
    oh = geo8[:, 0:4]
    v2, ew2, expl2 = _attn_core(
        geot_ref[...], oh, dt[:, 0:128], dt[:, 128:256],
        _unpack_top(gs_ref[...]), _unpack_bot(gs_ref[...]),
        db[:, 0:128], Wr2[...], R[...], T[...], gk[...], bk[...], W2k[...],
        b2k[...], gv[...], bv[...], W2v[...], b2v[...], ewb[...], HS[...])
    w2 = expl2 * (v2 * ew2)
    relp = geo8[:, 4:8]
    rows64 = (jnp.dot(w2, T4[...], preferred_element_type=jnp.float32)
              * jnp.dot(relp, M4[...], preferred_element_type=jnp.float32)
              + jnp.dot(expl2, E4[...], preferred_element_type=jnp.float32))
    rows_ref[...] = jnp.concatenate(
        [rows64, jnp.zeros_like(rows64)], axis=-1)


def _node2_body(Sn1_ref, Sn2_ref, Sd1_ref, Sd2_ref, h_ref, xm_ref,
                W1noA, W1noB, b1no, gno, blno, W2no, b2no,
                Wq1, bq1, gq, blq, Wq2, bq2, WdP, bPd, WsP,
                hnew_ref, td_ref, ts_ref):
    h = h_ref[...]
    num = Sn1_ref[0] + Sn1_ref[1] + Sn2_ref[0] + Sn2_ref[1]
    den = Sd1_ref[0] + Sd1_ref[1] + Sd2_ref[0] + Sd2_ref[1]
    out_attn = num / (den + 1e-16)
    y = (jnp.dot(out_attn, W1noA[...], preferred_element_type=jnp.float32)
         + jnp.dot(h, W1noB[...], preferred_element_type=jnp.float32) + b1no[...])
    out = jnp.dot(_ln_relu(y, gno[...], blno[...]), W2no[...],
                  preferred_element_type=jnp.float32) + b2no[...]
    h_new = out + h
    hnew_ref[...] = h_new
    q2 = _mlp(h_new, Wq1[...], bq1[...], gq[...], blq[...], Wq2[...], bq2[...])
    Pd = jnp.dot(h_new, WdP[...], preferred_element_type=jnp.float32) + bPd[...]
    Ps = jnp.dot(h_new, WsP[...], preferred_element_type=jnp.float32)
    z = jnp.zeros_like(q2)
    td_ref[...] = _pack2(Pd, jnp.concatenate([q2, z], axis=-1))
    ts_ref[...] = _pack2(Ps[:, 0:128], Ps[:, 128:256])


def _node3_body(U1_ref, U2_ref, xm_ref, xout_ref):
    U = U1_ref[0] + U1_ref[1] + U2_ref[0] + U2_ref[1]
    inv = 1.0 / (U[:, 48:64] + 1e-16)
    c = [jnp.mean(U[:, 16 * t:16 * t + 16] * inv, -1, keepdims=True)
         for t in range(3)]
    xm = xm_ref[...]
    mf = xm[:, 3:4]
    xnew = xm[:, 0:3] + jnp.concatenate(c, axis=-1) * mf
    pad = jnp.zeros((xnew.shape[0], 13), jnp.float32)
    xout_ref[...] = jnp.concatenate([xnew, pad], axis=-1)


def _full(a):
    return pl.BlockSpec(a.shape, lambda i: tuple(0 for _ in a.shape))


def _call_node1(h, xm, ws):
    n = h.shape[0]
    grid = (n // _TN,)
    ins = [pl.BlockSpec((_TN, 128), lambda i: (i, 0)),
           pl.BlockSpec((_TN, 16), lambda i: (i, 0))] + [_full(w) for w in ws]
    outs = [pl.BlockSpec((_TN, 256), lambda i: (i, 0)),
            pl.BlockSpec((_TN, 256), lambda i: (i, 0))]
    return pl.pallas_call(
        _node1_body, grid=grid, in_specs=ins, out_specs=outs,
        out_shape=[jax.ShapeDtypeStruct((n, 256), jnp.int32),
                   jax.ShapeDtypeStruct((n, 256), jnp.int32)],
    )(h, xm, *ws)


def _call_edge1(Gd, Gs, ws):
    e = Gd.shape[0]
    grid = (e // _TE,)
    ins = [pl.BlockSpec((_TE, 256), lambda i: (i, 0)),
           pl.BlockSpec((_TE, 256), lambda i: (i, 0))] + [_full(w) for w in ws]
    outs = [pl.BlockSpec((_TE, 128), lambda i: (i, 0)),
            pl.BlockSpec((_TE, 128), lambda i: (i, 0)),
            pl.BlockSpec((_TE, 8), lambda i: (i, 0)),
            pl.BlockSpec((NG, _TE), lambda i: (0, i))]
    return pl.pallas_call(
        _edge1_body, grid=grid, in_specs=ins, out_specs=outs,
        out_shape=[jax.ShapeDtypeStruct((e, 128), jnp.float32),
                   jax.ShapeDtypeStruct((e, 128), jnp.float32),
                   jax.ShapeDtypeStruct((e, 8), jnp.float32),
                   jax.ShapeDtypeStruct((NG, e), jnp.float32)],
    )(Gd, Gs, *ws)


def _call_edge2(Gd, Gs, geo8, geot, ws):
    e = Gd.shape[0]
    grid = (e // _TE,)
    ins = [pl.BlockSpec((_TE, 256), lambda i: (i, 0)),
           pl.BlockSpec((_TE, 128), lambda i: (i, 0)),
           pl.BlockSpec((_TE, 8), lambda i: (i, 0)),
           pl.BlockSpec((NG, _TE), lambda i: (0, i))] + [_full(w) for w in ws]
    outs = pl.BlockSpec((_TE, 128), lambda i: (i, 0))
    return pl.pallas_call(
        _edge2_body, grid=grid, in_specs=ins, out_specs=outs,
        out_shape=jax.ShapeDtypeStruct((e, 128), jnp.float32),
    )(Gd, Gs, geo8, geot, *ws)


def _call_node2(Sn1, Sn2, Sd1, Sd2, h, xm, ws):
    n = h.shape[0]
    grid = (n // _TN,)
    ins = [pl.BlockSpec((2, _TN, 128), lambda i: (0, i, 0)),
           pl.BlockSpec((2, _TN, 128), lambda i: (0, i, 0)),
           pl.BlockSpec((2, _TN, 128), lambda i: (0, i, 0)),
           pl.BlockSpec((2, _TN, 128), lambda i: (0, i, 0)),
           pl.BlockSpec((_TN, 128), lambda i: (i, 0)),
           pl.BlockSpec((_TN, 16), lambda i: (i, 0))] + [_full(w) for w in ws]
    outs = [pl.BlockSpec((_TN, 128), lambda i: (i, 0)),
            pl.BlockSpec((_TN, 256), lambda i: (i, 0)),
            pl.BlockSpec((_TN, 128), lambda i: (i, 0))]
    return pl.pallas_call(
        _node2_body, grid=grid, in_specs=ins, out_specs=outs,
        out_shape=[jax.ShapeDtypeStruct((n, 128), jnp.float32),
                   jax.ShapeDtypeStruct((n, 256), jnp.int32),
                   jax.ShapeDtypeStruct((n, 128), jnp.int32)],
    )(Sn1, Sn2, Sd1, Sd2, h, xm, *ws)


def _call_node3(U1, U2, xm):
    n = xm.shape[0]
    grid = (n // _TN,)
    ins = [pl.BlockSpec((2, _TN, 128), lambda i: (0, i, 0)),
           pl.BlockSpec((2, _TN, 128), lambda i: (0, i, 0)),
           pl.BlockSpec((_TN, 16), lambda i: (i, 0))]
    outs = pl.BlockSpec((_TN, 16), lambda i: (i, 0))
    return pl.pallas_call(
        _node3_body, grid=grid, in_specs=ins, out_specs=outs,
        out_shape=jax.ShapeDtypeStruct((n, 16), jnp.float32),
    )(U1, U2, xm)


def _sc_gather(table, idx):
    eg = idx.shape[0]
    d = table.shape[1]
    mesh = plsc.VectorSubcoreMesh(core_axis_name="core", subcore_axis_name="subcore")

    @functools.partial(
        pl.kernel,
        out_type=jax.ShapeDtypeStruct((eg, d), table.dtype),
        mesh=mesh)
    def k(t_hbm, i_hbm, o_hbm):
        def body(i_vmem, o_vmem):
            pltpu.sync_copy(t_hbm.at[i_vmem.at[0]], o_vmem)

        pltpu.emit_pipeline(
            body,
            grid=(eg // _SCW,),
            in_specs=[pl.BlockSpec((1, _SCW), lambda i: (0, i))],
            out_specs=[pl.BlockSpec((_SCW, d), lambda i: (i, 0))],
            core_axis_name=("core", "subcore"),
            dimension_semantics=(pltpu.PARALLEL,),
        )(i_hbm, o_hbm)

    return k(table, idx.reshape(1, eg))


def _sc_scatter_add(rows_list, idx, num_nodes):
    nrow = idx.shape[0] // _SCW
    assert len(rows_list) == 1
    d = rows_list[0].shape[1]
    base, rem = nrow // 32, nrow % 32
    rpt = (num_nodes // 16) & ~7
    tail = num_nodes - 16 * rpt
    mesh = plsc.VectorSubcoreMesh(core_axis_name="core", subcore_axis_name="subcore")
    zeros = jnp.zeros((num_nodes, d), jnp.float32)

    scratch = [pltpu.VMEM((2, _SCW), jnp.int32),
               pltpu.VMEM((2, _SCW, d), jnp.float32),
               pltpu.VMEM_SHARED((num_nodes, d), jnp.float32),
               pltpu.SemaphoreType.DMA((2,)),
               pltpu.SemaphoreType.DMA((2,))]

    @functools.partial(
        pl.kernel,
        out_type=[jax.ShapeDtypeStruct((2, num_nodes, d), jnp.float32)],
        mesh=mesh,
        scratch_types=scratch)
    def k(z_hbm, r_hbm, i_hbm, o_hbm, idxb, rowb, sh, isem, rsem):
        cid = lax.axis_index("core")
        sid = lax.axis_index("subcore")
        wid = sid * 2 + cid

        @pl.when(sid == 0)
        def _():
            pltpu.sync_copy(z_hbm, sh)

        plsc.subcore_barrier()

        start = wid * base + jnp.minimum(wid, rem)
        cnt = base + jnp.where(wid < rem, 1, 0)

        def fetch(b, j):
            off = pl.multiple_of((start + j) * _SCW, _SCW)
            pltpu.async_copy(i_hbm.at[pl.ds(off, _SCW)], idxb.at[b], isem.at[b])
            pltpu.async_copy(r_hbm.at[pl.ds(off, _SCW)], rowb.at[b], rsem.at[b])

        def drain(b):
            pltpu.make_async_copy(i_hbm.at[pl.ds(0, _SCW)], idxb.at[b],
                                  isem.at[b]).wait()
            pltpu.make_async_copy(r_hbm.at[pl.ds(0, _SCW)], rowb.at[b],
                                  rsem.at[b]).wait()

        for b in range(2):
            @pl.when(cnt > b)
            def _(b=b):
                fetch(b, b)

        @pl.loop(0, cnt, step=2)
        def _(t):
            for b in range(2):
                @pl.when(t + b < cnt)
                def _(b=b):
                    drain(b)
                    pltpu.sync_copy(rowb.at[b], sh.at[idxb.at[b]], add=True)

                    @pl.when(t + b + 2 < cnt)
                    def _(b=b):
                        fetch(b, t + b + 2)

        plsc.subcore_barrier()
        ns = pl.multiple_of(sid * rpt, 8)
        pltpu.sync_copy(sh.at[pl.ds(ns, rpt)], o_hbm.at[cid, pl.ds(ns, rpt)])
        if tail:
            @pl.when(sid == 0)
            def _():
                pltpu.sync_copy(sh.at[pl.ds(16 * rpt, tail)],
                                o_hbm.at[cid, pl.ds(16 * rpt, tail)])

    return k(zeros, rows_list[0], idx)


def _r_weight(W1):
    return jnp.concatenate([W1[t:RF:4, :] for t in range(4)], axis=1)


def _row(v):
    return v.reshape(1, -1)


def _prep_attn_weights(pk, pv, pew):
    Wr2 = jnp.concatenate([pk['W1'][:RF], pv['W1'][:RF], pew['W']], axis=1)
    gidx = jnp.arange(RF) // 4
    tidx = jnp.arange(RF) % 4
    R = (jnp.arange(NG)[:, None] == gidx[None, :]).astype(jnp.float32)
    T = (jnp.arange(4)[:, None] == tidx[None, :]).astype(jnp.float32)
    HS = (jnp.arange(128)[:, None] // HD
          == jnp.arange(NH)[None, :]).astype(jnp.float32)
    EX = HS.T
    return [Wr2, R, T, _row(pk['g']), _row(pk['bln']), pk['W2'], _row(pk['b2']),
            _row(pv['g']), _row(pv['bln']), pv['W2'], _row(pv['b2']),
            pew['b'].reshape(1, 1), HS, EX]


def _proj_weights(pk, pv):
    WdP = jnp.concatenate([pk['W1'][RF:RF + HID], pv['W1'][RF:RF + HID]], axis=1)
    bPd = _row(jnp.concatenate([pk['b1'], pv['b1']]))
    WsP = jnp.concatenate([pk['W1'][RF + HID:], pv['W1'][RF + HID:]], axis=1)
    return WdP, bPd, WsP


def _mlp_weights(p):
    return [p['W1'], _row(p['b1']), _row(p['g']), _row(p['bln']),
            p['W2'], _row(p['b2'])]


def kernel(h, x, params, edge_index, mask_ligand):
    n = h.shape[0]
    e = edge_index.shape[1]
    src = edge_index[0].astype(jnp.int32)
    dst = edge_index[1].astype(jnp.int32)
    mf = mask_ligand.astype(jnp.float32)
    xm = jnp.concatenate(
        [x, mf[:, None], jnp.zeros((n, 12), jnp.float32)], axis=-1)
    ec = e // 2
    srcs = [src[:ec], src[ec:]]
    dsts = [dst[:ec], dst[ec:]]

    WdP1, bPd1, WsP1 = _proj_weights(params['hk'], params['hv'])
    td1, ts1 = _call_node1(
        h, xm, _mlp_weights(params['hq']) + [WdP1, bPd1, WsP1])
    ew1 = _prep_attn_weights(params['hk'], params['hv'], params['ew_h'])
    Sns, Sds, geo8s, geots = [], [], [], []
    for c in range(2):
        Gd = _sc_gather(td1, dsts[c])
        Gs = _sc_gather(ts1, srcs[c])
        wv, ex, geo8, geot = _call_edge1(Gd, Gs, ew1)
        geo8s.append(geo8)
        geots.append(geot)
        (Sn,) = _sc_scatter_add([wv], dsts[c], n)
        (Sd,) = _sc_scatter_add([ex], dsts[c], n)
        Sns.append(Sn)
        Sds.append(Sd)

    pno = params['node_out']
    WdP2, bPd2, WsP2 = _proj_weights(params['xk'], params['xv'])
    ws2 = [pno['W1'][:HID], pno['W1'][HID:], _row(pno['b1']), _row(pno['g']),
           _row(pno['bln']), pno['W2'], _row(pno['b2'])] \
        + _mlp_weights(params['xq']) + [WdP2, bPd2, WsP2]
    h_new, td2, ts2 = _call_node2(Sns[0], Sns[1], Sds[0], Sds[1], h, xm, ws2)

    ew2 = _prep_attn_weights(params['xk'], params['xv'], params['ew_x'])
    j16 = jnp.arange(16)
    j64 = jnp.arange(64)
    T4 = ((j16[:, None] == j64[None, :] % 16) & (j64[None, :] < 48)).astype(jnp.float32)
    M4 = ((jnp.arange(4)[:, None] == j64[None, :] // 16)
          & (j64[None, :] < 48)).astype(jnp.float32)
    E4 = (j16[:, None] + 48 == j64[None, :]).astype(jnp.float32)
    ew2 = ew2[:3] + [T4, M4, E4] + ew2[3:]
    Us = []
    for c in range(2):
        Gd = _sc_gather(td2, dsts[c])
        Gs = _sc_gather(ts2, srcs[c])
        rows2 = _call_edge2(Gd, Gs, geo8s[c], geots[c], ew2)
        (U,) = _sc_scatter_add([rows2], dsts[c], n)
        Us.append(U)
    xout = _call_node3(Us[0], Us[1], xm)
    return h_new, xout[:, :3]
